# baseline (speedup 1.0000x reference)
import jax
import jax.numpy as jnp
from jax.experimental import pallas as pl


def kernel(features, cluster_features, xws, yws, raw_affine, edge_index, params):
    src = edge_index[0]
    dst = edge_index[1]
    num_nodes = features.shape[0]
    x = jnp.concatenate([features, cluster_features], axis=1)
    for i in range(4):
        W = params['conv%d_W' % i]; b = params['conv%d_b' % i]
        a_s = params['conv%d_asrc' % i]; a_d = params['conv%d_adst' % i]
        z = x @ W
        el = z @ a_s
        er = z @ a_d
        e = jax.nn.leaky_relu(el[src] + er[dst], 0.2)
        emax = jax.ops.segment_max(e, dst, num_segments=num_nodes)
        ex = jnp.exp(e - emax[dst])
        denom = jax.ops.segment_sum(ex, dst, num_segments=num_nodes)
        alpha = ex / denom[dst]
        msg = z[src] * alpha[:, None]
        agg = jax.ops.segment_sum(msg, dst, num_segments=num_nodes)
        x = jax.nn.relu(agg + b)
    conv_features = x

    def prelu(v, a):
        return jnp.where(v >= 0, v, a * v)

    src_feat = conv_features @ params['src_W'] + params['src_b']
    dst_feat = conv_features @ params['dst_W'] + params['dst_b']
    feat_cat = jnp.concatenate([src_feat[src], xws[src], yws[src], dst_feat[dst], xws[dst], yws[dst]], axis=1)
    h = prelu(feat_cat, params['prelu1'])
    h = h @ params['cls1_W'] + params['cls1_b']
    h = prelu(h, params['prelu2'])
    pred_conn = h @ params['cls2_W'] + params['cls2_b']
    prob = jax.nn.sigmoid(pred_conn)[:, 0]
    msg_den = raw_affine * (prob - (1.0 - prob))
    deg = jax.ops.segment_sum(jnp.ones_like(msg_den), src, num_segments=num_nodes)
    pred_den = jax.ops.segment_sum(msg_den, src, num_segments=num_nodes) / jnp.maximum(deg, 1.0)
    return pred_conn, pred_den


# speedup vs baseline: 8.9252x; 8.9252x over previous
import jax
import jax.numpy as jnp
from jax import lax
from jax.experimental import pallas as pl
from jax.experimental.pallas import tpu as pltpu
from jax.experimental.pallas import tpu_sc as plsc

N = 10000
NP = 10240
E = 160000
EP = 163840
EPT = EP // 16
ETG = EP // 32
NCHUNK = EPT // 128
NROWS_T = NP // 16

_MESH = plsc.VectorSubcoreMesh(
    core_axis_name="c", subcore_axis_name="s", num_cores=2, num_subcores=16)
_SC_PARAMS = pltpu.CompilerParams(needs_layout_passes=False,
                                  use_tc_tiling_on_sc=False)


def _prelu(v, a):
    return jnp.where(v >= 0, v, a * v)


def _gat_linear(parts, wparts, bparts, a_s, a_d, relu_in):
    nparts = len(parts)
    dps = [p.shape[1] for p in parts]
    dout = wparts[0].shape[1]
    h4 = dout // 4
    blk = 1024

    def body(*refs):
        part_refs = refs[:nparts]
        w_refs = refs[nparts:2 * nparts]
        b_refs = refs[2 * nparts:3 * nparts]
        as_ref = refs[3 * nparts]
        ad_ref = refs[3 * nparts + 1]
        zq_refs = refs[3 * nparts + 2:3 * nparts + 6]
        el_ref = refs[3 * nparts + 6]
        er_ref = refs[3 * nparts + 7]
        z = None
        for p_ref, w_ref, b_ref in zip(part_refs, w_refs, b_refs):
            a = p_ref[...]
            if relu_in:
                a = jnp.maximum(a + b_ref[...], 0.0)
            d = jnp.dot(a, w_ref[...], preferred_element_type=jnp.float32)
            z = d if z is None else z + d
        for q in range(4):
            zq_refs[q][...] = z[:, q * h4:(q + 1) * h4]
        el_ref[...] = jnp.dot(z, as_ref[...], preferred_element_type=jnp.float32)
        er_ref[...] = jnp.dot(z, ad_ref[...], preferred_element_type=jnp.float32)

    in_specs = (
        [pl.BlockSpec((blk, dp), lambda i: (i, 0)) for dp in dps]
        + [pl.BlockSpec((dp, dout), lambda i: (0, 0)) for dp in dps]
        + [pl.BlockSpec((1, dp), lambda i: (0, 0)) for dp in dps]
        + [pl.BlockSpec((dout, 1), lambda i: (0, 0))] * 2
    )
    out_specs = ([pl.BlockSpec((blk, h4), lambda i: (i, 0))] * 4
                 + [pl.BlockSpec((blk, 1), lambda i: (i, 0))] * 2)
    out_shape = ([jax.ShapeDtypeStruct((NP, h4), jnp.float32)] * 4
                 + [jax.ShapeDtypeStruct((NP, 1), jnp.float32)] * 2)
    return pl.pallas_call(
        body,
        grid=(NP // blk,),
        in_specs=in_specs,
        out_specs=out_specs,
        out_shape=out_shape,
    )(*parts, *wparts, *bparts, a_s, a_d)


def _sc_gat_edge(zq, el, er, srcp, dstp):
    h4 = zq[0].shape[1]

    def body(z0_hbm, z1_hbm, z2_hbm, z3_hbm, el_hbm, er_hbm, src_hbm, dst_hbm,
             a0_hbm, a1_hbm, a2_hbm, a3_hbm,
             el_v, er_v, src_v, dst_v, alpha_v, denom_v, rows_v, idx_v,
             stage_v, vmax_v, gmax_sh, denom_sh, acc_sh, sem):
        c = lax.axis_index("c")
        s = lax.axis_index("s")
        base = s * EPT
        zeros16 = jnp.zeros((16,), jnp.float32)
        iota16 = lax.iota(jnp.int32, 16)

        pltpu.sync_copy(el_hbm, el_v)
        pltpu.sync_copy(er_hbm, er_v)
        pltpu.sync_copy(src_hbm.at[pl.ds(base, EPT)], src_v)
        pltpu.sync_copy(dst_hbm.at[pl.ds(base, EPT)], dst_v)

        def zden(i, _):
            denom_v[pl.ds(i * 16, 16)] = zeros16
            return 0
        lax.fori_loop(0, NP // 16, zden, 0)
        pltpu.sync_copy(denom_v.at[pl.ds(0, NROWS_T)],
                        denom_sh.at[pl.ds(s * NROWS_T, NROWS_T)])

        def p1(i, vmax):
            si = src_v[pl.ds(i * 16, 16)]
            di = dst_v[pl.ds(i * 16, 16)]
            e = plsc.load_gather(el_v, [si]) + plsc.load_gather(er_v, [di])
            e = jnp.where(e >= 0, e, 0.2 * e)
            return jnp.maximum(vmax, e)
        vmax = lax.fori_loop(0, EPT // 16, p1,
                             jnp.full((16,), -3.0e38, jnp.float32))
        vmax_v[...] = vmax
        pltpu.sync_copy(vmax_v, gmax_sh.at[s])
        plsc.subcore_barrier()
        pltpu.sync_copy(gmax_sh, stage_v)
        m = stage_v[0]
        for t in range(1, 16):
            m = jnp.maximum(m, stage_v[t])
        gmax = jnp.max(m)

        def p2(i, _):
            si = src_v[pl.ds(i * 16, 16)]
            di = dst_v[pl.ds(i * 16, 16)]
            e = plsc.load_gather(el_v, [si]) + plsc.load_gather(er_v, [di])
            e = jnp.where(e >= 0, e, 0.2 * e)
            gid = base + i * 16 + iota16
            ex = jnp.exp(e - gmax)
            ex = jnp.where(gid < E, ex, 0.0)
            alpha_v[pl.ds(i * 16, 16)] = ex
            plsc.addupdate_scatter(denom_v, [di], ex)
            return 0
        lax.fori_loop(0, EPT // 16, p2, 0)
        plsc.subcore_barrier()

        def dred(j, _):
            for k in range(8):
                idx_v[0, pl.ds(k * 16, 16)] = j * 128 + k * 16 + iota16
            pltpu.sync_copy(denom_v.at[pl.ds(j * 128, 128)],
                            denom_sh.at[idx_v.at[0]], add=True)
            return 0
        lax.fori_loop(0, NP // 128, dred, 0)
        plsc.subcore_barrier()
        pltpu.sync_copy(denom_sh, denom_v)

        def p3(i, _):
            di = dst_v[pl.ds(i * 16, 16)]
            den = plsc.load_gather(denom_v, [di])
            alpha_v[pl.ds(i * 16, 16)] = (
                alpha_v[pl.ds(i * 16, 16)] / jnp.maximum(den, 1e-30))
            return 0
        lax.fori_loop(0, EPT // 16, p3, 0)

        def phase_b(z_hbm, agg_hbm):
            def zrow(i, _):
                for k in range(h4 // 16):
                    rows_v[i, pl.ds(k * 16, 16)] = zeros16
                return 0
            lax.fori_loop(0, 128, zrow, 0)

            def zacc(j, _):
                pltpu.sync_copy(rows_v,
                                acc_sh.at[pl.ds(s * NROWS_T + j * 128, 128)])
                return 0
            lax.fori_loop(0, NROWS_T // 128, zacc, 0)
            plsc.subcore_barrier()

            def chunk(j, _):
                pltpu.async_copy(z_hbm.at[src_v.at[pl.ds(j * 128, 128)]],
                                 rows_v, sem).wait()

                def grp(g, _):
                    av = alpha_v[pl.ds(j * 128 + g * 16, 16)]
                    for r in range(16):
                        arow = jnp.full((16,), av[r], jnp.float32)
                        row = g * 16 + r
                        for k in range(h4 // 16):
                            rows_v[row, pl.ds(k * 16, 16)] = (
                                rows_v[row, pl.ds(k * 16, 16)] * arow)
                    return 0
                lax.fori_loop(0, 8, grp, 0)
                for k in range(8):
                    idx_v[0, pl.ds(k * 16, 16)] = (
                        dst_v[pl.ds(j * 128 + k * 16, 16)])
                pltpu.sync_copy(rows_v, acc_sh.at[idx_v.at[0]], add=True)
                return 0
            lax.fori_loop(0, NCHUNK, chunk, 0)
            plsc.subcore_barrier()
            pltpu.sync_copy(acc_sh.at[pl.ds(s * NROWS_T, NROWS_T)],
                            agg_hbm.at[pl.ds(s * NROWS_T, NROWS_T)])

        @pl.when(c == 0)
        def _():
            phase_b(z0_hbm, a0_hbm)
            phase_b(z1_hbm, a1_hbm)

        @pl.when(c == 1)
        def _():
            phase_b(z2_hbm, a2_hbm)
            phase_b(z3_hbm, a3_hbm)

    run = pl.kernel(
        body,
        compiler_params=_SC_PARAMS,
        out_type=tuple(jax.ShapeDtypeStruct((NP, h4), jnp.float32)
                       for _ in range(4)),
        mesh=_MESH,
        scratch_types=[
            pltpu.VMEM((NP,), jnp.float32),
            pltpu.VMEM((NP,), jnp.float32),
            pltpu.VMEM((EPT,), jnp.int32),
            pltpu.VMEM((EPT,), jnp.int32),
            pltpu.VMEM((EPT,), jnp.float32),
            pltpu.VMEM((NP,), jnp.float32),
            pltpu.VMEM((128, h4), jnp.float32),
            pltpu.VMEM((1, 128), jnp.int32),
            pltpu.VMEM((16, 16), jnp.float32),
            pltpu.VMEM((16,), jnp.float32),
            pltpu.VMEM_SHARED((16, 16), jnp.float32),
            pltpu.VMEM_SHARED((NP,), jnp.float32),
            pltpu.VMEM_SHARED((NP, h4), jnp.float32),
            pltpu.SemaphoreType.DMA,
        ],
    )
    return run(*zq, el, er, srcp, dstp)


def _tables(parts, bparts, swparts, dwparts, sxyb, dxyb):
    nparts = len(parts)
    dps = [p.shape[1] for p in parts]
    blk = 1024

    def body(*refs):
        part_refs = refs[:nparts]
        b_refs = refs[nparts:2 * nparts]
        sw_refs = refs[2 * nparts:3 * nparts]
        dw_refs = refs[3 * nparts:4 * nparts]
        sxy_ref = refs[4 * nparts]
        dxy_ref = refs[4 * nparts + 1]
        st_ref = refs[4 * nparts + 2]
        dt_ref = refs[4 * nparts + 3]
        st = sxy_ref[...]
        dt = dxy_ref[...]
        for p_ref, b_ref, sw_ref, dw_ref in zip(part_refs, b_refs, sw_refs,
                                                dw_refs):
            a = jnp.maximum(p_ref[...] + b_ref[...], 0.0)
            st = st + jnp.dot(a, sw_ref[...], preferred_element_type=jnp.float32)
            dt = dt + jnp.dot(a, dw_ref[...], preferred_element_type=jnp.float32)
        st_ref[...] = st
        dt_ref[...] = dt

    in_specs = (
        [pl.BlockSpec((blk, dp), lambda i: (i, 0)) for dp in dps]
        + [pl.BlockSpec((1, dp), lambda i: (0, 0)) for dp in dps]
        + [pl.BlockSpec((dp, 16), lambda i: (0, 0)) for dp in dps] * 2
        + [pl.BlockSpec((blk, 16), lambda i: (i, 0))] * 2
    )
    return pl.pallas_call(
        body,
        grid=(NP // blk,),
        in_specs=in_specs,
        out_specs=[pl.BlockSpec((blk, 16), lambda i: (i, 0))] * 2,
        out_shape=[jax.ShapeDtypeStruct((NP, 16), jnp.float32)] * 2,
    )(*parts, *bparts, *swparts, *dwparts, sxyb, dxyb)


def _sc_gather_tables(st, dt, srcp, dstp):
    def body(st_hbm, dt_hbm, src_hbm, dst_hbm, fa_hbm, fb_hbm,
             src_v, dst_v, rows_a, rows_b, sem):
        c = lax.axis_index("c")
        s = lax.axis_index("s")
        base = (c * 16 + s) * ETG
        pltpu.sync_copy(src_hbm.at[pl.ds(base, ETG)], src_v)
        pltpu.sync_copy(dst_hbm.at[pl.ds(base, ETG)], dst_v)

        def chunk(j, _):
            pltpu.async_copy(st_hbm.at[src_v.at[pl.ds(j * 128, 128)]],
                             rows_a, sem).wait()
            pltpu.sync_copy(rows_a, fa_hbm.at[pl.ds(base + j * 128, 128)])
            pltpu.async_copy(dt_hbm.at[dst_v.at[pl.ds(j * 128, 128)]],
                             rows_b, sem).wait()
            pltpu.sync_copy(rows_b, fb_hbm.at[pl.ds(base + j * 128, 128)])
            return 0
        lax.fori_loop(0, ETG // 128, chunk, 0)

    run = pl.kernel(
        body,
        compiler_params=_SC_PARAMS,
        out_type=(jax.ShapeDtypeStruct((EP, 16), jnp.float32),
                  jax.ShapeDtypeStruct((EP, 16), jnp.float32)),
        mesh=_MESH,
        scratch_types=[
            pltpu.VMEM((ETG,), jnp.int32),
            pltpu.VMEM((ETG,), jnp.int32),
            pltpu.VMEM((128, 16), jnp.float32),
            pltpu.VMEM((128, 16), jnp.float32),
            pltpu.SemaphoreType.DMA,
        ],
    )
    return run(st, dt, srcp, dstp)


def _mlp(fa, fb, rap, w1a, w1b, b1, p1a, p1b, p2, w2, b2):
    blk = 8192

    def body(fa_ref, fb_ref, ra_ref, w1a_ref, w1b_ref, b1_ref, p1a_ref,
             p1b_ref, p2_ref, w2_ref, b2_ref, pc_ref, md_ref):
        ha = _prelu(fa_ref[...], p1a_ref[...])
        hb = _prelu(fb_ref[...], p1b_ref[...])
        h = (jnp.dot(ha, w1a_ref[...], preferred_element_type=jnp.float32)
             + jnp.dot(hb, w1b_ref[...], preferred_element_type=jnp.float32)
             + b1_ref[...])
        h = _prelu(h, p2_ref[...])
        pc = jnp.dot(h, w2_ref[...], preferred_element_type=jnp.float32) + b2_ref[...]
        pc_ref[...] = pc
        md_ref[...] = ra_ref[...] * (2.0 * jax.nn.sigmoid(pc) - 1.0)

    return pl.pallas_call(
        body,
        grid=(EP // blk,),
        in_specs=[
            pl.BlockSpec((blk, 16), lambda i: (i, 0)),
            pl.BlockSpec((blk, 16), lambda i: (i, 0)),
            pl.BlockSpec((blk, 1), lambda i: (i, 0)),
            pl.BlockSpec((16, 8), lambda i: (0, 0)),
            pl.BlockSpec((16, 8), lambda i: (0, 0)),
            pl.BlockSpec((1, 8), lambda i: (0, 0)),
            pl.BlockSpec((1, 16), lambda i: (0, 0)),
            pl.BlockSpec((1, 16), lambda i: (0, 0)),
            pl.BlockSpec((1, 8), lambda i: (0, 0)),
            pl.BlockSpec((8, 1), lambda i: (0, 0)),
            pl.BlockSpec((1, 1), lambda i: (0, 0)),
        ],
        out_specs=[
            pl.BlockSpec((blk, 1), lambda i: (i, 0)),
            pl.BlockSpec((blk, 1), lambda i: (i, 0)),
        ],
        out_shape=[
            jax.ShapeDtypeStruct((EP, 1), jnp.float32),
            jax.ShapeDtypeStruct((EP, 1), jnp.float32),
        ],
    )(fa, fb, rap, w1a, w1b, b1, p1a, p1b, p2, w2, b2)


def _sc_segmean_src(md, srcp):
    def body(md_hbm, src_hbm, out_hbm,
             msg_v, src_v, den_v, deg_v, idx_v, den_sh, deg_sh):
        c = lax.axis_index("c")
        s = lax.axis_index("s")

        @pl.when(c == 0)
        def _():
            base = s * EPT
            zeros16 = jnp.zeros((16,), jnp.float32)
            iota16 = lax.iota(jnp.int32, 16)
            pltpu.sync_copy(md_hbm.at[pl.ds(base, EPT)], msg_v)
            pltpu.sync_copy(src_hbm.at[pl.ds(base, EPT)], src_v)

            def zero(i, _):
                den_v[pl.ds(i * 16, 16)] = zeros16
                deg_v[pl.ds(i * 16, 16)] = zeros16
                return 0
            lax.fori_loop(0, NP // 16, zero, 0)
            pltpu.sync_copy(den_v.at[pl.ds(0, NROWS_T)],
                            den_sh.at[pl.ds(s * NROWS_T, NROWS_T)])
            pltpu.sync_copy(deg_v.at[pl.ds(0, NROWS_T)],
                            deg_sh.at[pl.ds(s * NROWS_T, NROWS_T)])

            def p(i, _):
                si = src_v[pl.ds(i * 16, 16)]
                mdv = msg_v[pl.ds(i * 16, 16)]
                gid = base + i * 16 + iota16
                w = jnp.where(gid < E, 1.0, 0.0)
                plsc.addupdate_scatter(den_v, [si], mdv)
                plsc.addupdate_scatter(deg_v, [si], w)
                return 0
            lax.fori_loop(0, EPT // 16, p, 0)
            plsc.subcore_barrier()

            def dred(j, _):
                for k in range(8):
                    idx_v[0, pl.ds(k * 16, 16)] = j * 128 + k * 16 + iota16
                pltpu.sync_copy(den_v.at[pl.ds(j * 128, 128)],
                                den_sh.at[idx_v.at[0]], add=True)
                pltpu.sync_copy(deg_v.at[pl.ds(j * 128, 128)],
                                deg_sh.at[idx_v.at[0]], add=True)
                return 0
            lax.fori_loop(0, NP // 128, dred, 0)
            plsc.subcore_barrier()

            pltpu.sync_copy(den_sh.at[pl.ds(s * NROWS_T, NROWS_T)],
                            den_v.at[pl.ds(0, NROWS_T)])
            pltpu.sync_copy(deg_sh.at[pl.ds(s * NROWS_T, NROWS_T)],
                            deg_v.at[pl.ds(0, NROWS_T)])

            def fin(i, _):
                d = den_v[pl.ds(i * 16, 16)]
                g = deg_v[pl.ds(i * 16, 16)]
                den_v[pl.ds(i * 16, 16)] = d / jnp.maximum(g, 1.0)
                return 0
            lax.fori_loop(0, NROWS_T // 16, fin, 0)
            pltpu.sync_copy(den_v.at[pl.ds(0, NROWS_T)],
                            out_hbm.at[pl.ds(s * NROWS_T, NROWS_T)])

    run = pl.kernel(
        body,
        compiler_params=_SC_PARAMS,
        out_type=jax.ShapeDtypeStruct((NP,), jnp.float32),
        mesh=_MESH,
        scratch_types=[
            pltpu.VMEM((EPT,), jnp.float32),
            pltpu.VMEM((EPT,), jnp.int32),
            pltpu.VMEM((NP,), jnp.float32),
            pltpu.VMEM((NP,), jnp.float32),
            pltpu.VMEM((1, 128), jnp.int32),
            pltpu.VMEM_SHARED((NP,), jnp.float32),
            pltpu.VMEM_SHARED((NP,), jnp.float32),
        ],
    )
    return run(md, srcp)


def kernel(features, cluster_features, xws, yws, raw_affine, edge_index, params):
    src = edge_index[0]
    dst = edge_index[1]
    srcp = jnp.pad(src, (0, EP - E))
    dstp = jnp.pad(dst, (0, EP - E))

    parts = [jnp.pad(features, ((0, NP - N), (0, 0))),
             jnp.pad(cluster_features, ((0, NP - N), (0, 0)))]
    bparts = [jnp.zeros((1, 128), jnp.float32)] * 2
    relu_in = False

    for i in range(4):
        w = params['conv%d_W' % i]
        b = params['conv%d_b' % i]
        a_s = params['conv%d_asrc' % i][:, None]
        a_d = params['conv%d_adst' % i][:, None]
        npart = len(parts)
        dp = w.shape[0] // npart
        h4 = w.shape[1] // 4
        wparts = [w[k * dp:(k + 1) * dp] for k in range(npart)]
        *zq, el, er = _gat_linear(parts, wparts, bparts, a_s, a_d, relu_in)
        aggq = _sc_gat_edge(zq, el.reshape(NP), er.reshape(NP), srcp, dstp)
        parts = list(aggq)
        bparts = [b[k * h4:(k + 1) * h4][None, :] for k in range(4)]
        relu_in = True

    xy = jnp.zeros((NP, 16), jnp.float32)
    xy = xy.at[:N, 6].set(xws[:, 0]).at[:N, 7].set(yws[:, 0])
    sxyb = xy.at[:, 0:6].add(params['src_b'][None, :])
    dxyb = xy.at[:, 0:6].add(params['dst_b'][None, :])
    swp = jnp.zeros((128, 16), jnp.float32).at[:, 0:6].set(params['src_W'])
    dwp = jnp.zeros((128, 16), jnp.float32).at[:, 0:6].set(params['dst_W'])
    swparts = [swp[k * 32:(k + 1) * 32] for k in range(4)]
    dwparts = [dwp[k * 32:(k + 1) * 32] for k in range(4)]
    st, dt = _tables(parts, bparts, swparts, dwparts, sxyb, dxyb)

    fa, fb = _sc_gather_tables(st, dt, srcp, dstp)

    rap = jnp.pad(raw_affine, (0, EP - E))[:, None]
    w1 = params['cls1_W']
    w1a = jnp.zeros((16, 8), jnp.float32).at[0:8].set(w1[0:8])
    w1b = jnp.zeros((16, 8), jnp.float32).at[0:8].set(w1[8:16])
    p1 = params['prelu1']
    p1a = jnp.zeros((1, 16), jnp.float32).at[0, 0:8].set(p1[0:8])
    p1b = jnp.zeros((1, 16), jnp.float32).at[0, 0:8].set(p1[8:16])
    pcp, mdp = _mlp(fa, fb, rap, w1a, w1b, params['cls1_b'][None, :],
                    p1a, p1b, params['prelu2'][None, :],
                    params['cls2_W'], params['cls2_b'][None, :])

    pred_den = _sc_segmean_src(mdp.reshape(EP), srcp)
    return pcp[:E], pred_den[:N]


# speedup vs baseline: 11.5974x; 1.2994x over previous
import jax
import jax.numpy as jnp
from jax import lax
from jax.experimental import pallas as pl
from jax.experimental.pallas import tpu as pltpu
from jax.experimental.pallas import tpu_sc as plsc

N = 10000
NP = 10240
E = 160000
EP = 163840
EPT = EP // 16
ETG = EP // 32
NCHUNK = EPT // 128
NROWS_T = NP // 16

_MESH = plsc.VectorSubcoreMesh(
    core_axis_name="c", subcore_axis_name="s", num_cores=2, num_subcores=16)
_SC_PARAMS = pltpu.CompilerParams(needs_layout_passes=False,
                                  use_tc_tiling_on_sc=False)


def _prelu(v, a):
    return jnp.where(v >= 0, v, a * v)


def _gat_linear(parts, wparts, bparts, a_s, a_d, relu_in):
    nparts = len(parts)
    dps = [p.shape[1] for p in parts]
    dout = wparts[0].shape[1]
    h4 = dout // 4
    blk = 1024

    def body(*refs):
        part_refs = refs[:nparts]
        w_refs = refs[nparts:2 * nparts]
        b_refs = refs[2 * nparts:3 * nparts]
        as_ref = refs[3 * nparts]
        ad_ref = refs[3 * nparts + 1]
        zq_refs = refs[3 * nparts + 2:3 * nparts + 6]
        el_ref = refs[3 * nparts + 6]
        er_ref = refs[3 * nparts + 7]
        z = None
        for p_ref, w_ref, b_ref in zip(part_refs, w_refs, b_refs):
            a = p_ref[...]
            if relu_in:
                a = jnp.maximum(a + b_ref[...], 0.0)
            d = jnp.dot(a, w_ref[...], preferred_element_type=jnp.float32)
            z = d if z is None else z + d
        for q in range(4):
            zq_refs[q][...] = z[:, q * h4:(q + 1) * h4]
        el_ref[...] = jnp.dot(z, as_ref[...], preferred_element_type=jnp.float32)
        er_ref[...] = jnp.dot(z, ad_ref[...], preferred_element_type=jnp.float32)

    in_specs = (
        [pl.BlockSpec((blk, dp), lambda i: (i, 0)) for dp in dps]
        + [pl.BlockSpec((dp, dout), lambda i: (0, 0)) for dp in dps]
        + [pl.BlockSpec((1, dp), lambda i: (0, 0)) for dp in dps]
        + [pl.BlockSpec((dout, 1), lambda i: (0, 0))] * 2
    )
    out_specs = ([pl.BlockSpec((blk, h4), lambda i: (i, 0))] * 4
                 + [pl.BlockSpec((blk, 1), lambda i: (i, 0))] * 2)
    out_shape = ([jax.ShapeDtypeStruct((NP, h4), jnp.float32)] * 4
                 + [jax.ShapeDtypeStruct((NP, 1), jnp.float32)] * 2)
    return pl.pallas_call(
        body,
        grid=(NP // blk,),
        in_specs=in_specs,
        out_specs=out_specs,
        out_shape=out_shape,
    )(*parts, *wparts, *bparts, a_s, a_d)


def _sc_gat_edge(zq, el, er, srcp, dstp):
    h4 = zq[0].shape[1]

    def body(z0_hbm, z1_hbm, z2_hbm, z3_hbm, el_hbm, er_hbm, src_hbm, dst_hbm,
             a0_hbm, a1_hbm, a2_hbm, a3_hbm,
             el_v, er_v, src_v, dst_v, alpha_v, denom_v, rows_v, rows2_v,
             idx_v, stage_v, vmax_v, gmax_sh, denom_sh, acc_sh, sem, sem2):
        c = lax.axis_index("c")
        s = lax.axis_index("s")
        base = s * EPT
        zeros16 = jnp.zeros((16,), jnp.float32)
        iota16 = lax.iota(jnp.int32, 16)

        pltpu.sync_copy(el_hbm, el_v)
        pltpu.sync_copy(er_hbm, er_v)
        pltpu.sync_copy(src_hbm.at[pl.ds(base, EPT)], src_v)
        pltpu.sync_copy(dst_hbm.at[pl.ds(base, EPT)], dst_v)

        def zden(i, _):
            denom_v[pl.ds(i * 16, 16)] = zeros16
            return 0
        lax.fori_loop(0, NP // 16, zden, 0)
        pltpu.sync_copy(denom_v.at[pl.ds(0, NROWS_T)],
                        denom_sh.at[pl.ds(s * NROWS_T, NROWS_T)])

        def p1(i, vmax):
            si = src_v[pl.ds(i * 16, 16)]
            di = dst_v[pl.ds(i * 16, 16)]
            e = plsc.load_gather(el_v, [si]) + plsc.load_gather(er_v, [di])
            e = jnp.where(e >= 0, e, 0.2 * e)
            alpha_v[pl.ds(i * 16, 16)] = e
            return jnp.maximum(vmax, e)
        vmax = lax.fori_loop(0, EPT // 16, p1,
                             jnp.full((16,), -3.0e38, jnp.float32))
        vmax_v[...] = vmax
        pltpu.sync_copy(vmax_v, gmax_sh.at[s])
        plsc.subcore_barrier()
        pltpu.sync_copy(gmax_sh, stage_v)
        m = stage_v[0]
        for t in range(1, 16):
            m = jnp.maximum(m, stage_v[t])
        gmax = jnp.max(m)

        def p2(i, _):
            di = dst_v[pl.ds(i * 16, 16)]
            e = alpha_v[pl.ds(i * 16, 16)]
            gid = base + i * 16 + iota16
            ex = jnp.exp(e - gmax)
            ex = jnp.where(gid < E, ex, 0.0)
            alpha_v[pl.ds(i * 16, 16)] = ex
            plsc.addupdate_scatter(denom_v, [di], ex)
            return 0
        lax.fori_loop(0, EPT // 16, p2, 0)
        plsc.subcore_barrier()

        def dred(j, _):
            for k in range(8):
                idx_v[0, pl.ds(k * 16, 16)] = j * 128 + k * 16 + iota16
            pltpu.sync_copy(denom_v.at[pl.ds(j * 128, 128)],
                            denom_sh.at[idx_v.at[0]], add=True)
            return 0
        lax.fori_loop(0, NP // 128, dred, 0)
        plsc.subcore_barrier()
        pltpu.sync_copy(denom_sh, denom_v)

        def p3(i, _):
            di = dst_v[pl.ds(i * 16, 16)]
            den = plsc.load_gather(denom_v, [di])
            alpha_v[pl.ds(i * 16, 16)] = (
                alpha_v[pl.ds(i * 16, 16)] / jnp.maximum(den, 1e-30))
            return 0
        lax.fori_loop(0, EPT // 16, p3, 0)

        def phase_b(z_hbm, agg_hbm):
            def zrow(i, _):
                for k in range(h4 // 16):
                    rows_v[i, pl.ds(k * 16, 16)] = zeros16
                return 0
            lax.fori_loop(0, 128, zrow, 0)

            def zacc(j, _):
                pltpu.sync_copy(rows_v,
                                acc_sh.at[pl.ds(s * NROWS_T + j * 128, 128)])
                return 0
            lax.fori_loop(0, NROWS_T // 128, zacc, 0)
            plsc.subcore_barrier()

            bufs = (rows_v, rows2_v)
            sems = (sem, sem2)
            for b in range(2):
                pltpu.async_copy(z_hbm.at[src_v.at[pl.ds(b * 128, 128)]],
                                 bufs[b], sems[b])

            def outer(j2, _):
                for b in range(2):
                    j = j2 * 2 + b
                    buf = bufs[b]
                    sm = sems[b]
                    for k in range(8):
                        idx_v[0, pl.ds(k * 16, 16)] = (
                            dst_v[pl.ds(j * 128 + k * 16, 16)])
                    pltpu.make_async_copy(
                        z_hbm.at[src_v.at[pl.ds(j * 128, 128)]], buf,
                        sm).wait()

                    def grp(g, _):
                        av = alpha_v[pl.ds(j * 128 + g * 16, 16)]
                        for r in range(16):
                            arow = jnp.full((16,), av[r], jnp.float32)
                            row = g * 16 + r
                            for k in range(h4 // 16):
                                buf[row, pl.ds(k * 16, 16)] = (
                                    buf[row, pl.ds(k * 16, 16)] * arow)
                        return 0
                    lax.fori_loop(0, 8, grp, 0)
                    pltpu.sync_copy(buf, acc_sh.at[idx_v.at[0]], add=True)

                    @pl.when(j + 2 < NCHUNK)
                    def _():
                        pltpu.async_copy(
                            z_hbm.at[src_v.at[pl.ds((j + 2) * 128, 128)]],
                            buf, sm)
                return 0
            lax.fori_loop(0, NCHUNK // 2, outer, 0)
            plsc.subcore_barrier()
            pltpu.sync_copy(acc_sh.at[pl.ds(s * NROWS_T, NROWS_T)],
                            agg_hbm.at[pl.ds(s * NROWS_T, NROWS_T)])

        @pl.when(c == 0)
        def _():
            phase_b(z0_hbm, a0_hbm)
            phase_b(z1_hbm, a1_hbm)

        @pl.when(c == 1)
        def _():
            phase_b(z2_hbm, a2_hbm)
            phase_b(z3_hbm, a3_hbm)

    run = pl.kernel(
        body,
        compiler_params=_SC_PARAMS,
        out_type=tuple(jax.ShapeDtypeStruct((NP, h4), jnp.float32)
                       for _ in range(4)),
        mesh=_MESH,
        scratch_types=[
            pltpu.VMEM((NP,), jnp.float32),
            pltpu.VMEM((NP,), jnp.float32),
            pltpu.VMEM((EPT,), jnp.int32),
            pltpu.VMEM((EPT,), jnp.int32),
            pltpu.VMEM((EPT,), jnp.float32),
            pltpu.VMEM((NP,), jnp.float32),
            pltpu.VMEM((128, h4), jnp.float32),
            pltpu.VMEM((128, h4), jnp.float32),
            pltpu.VMEM((1, 128), jnp.int32),
            pltpu.VMEM((16, 16), jnp.float32),
            pltpu.VMEM((16,), jnp.float32),
            pltpu.VMEM_SHARED((16, 16), jnp.float32),
            pltpu.VMEM_SHARED((NP,), jnp.float32),
            pltpu.VMEM_SHARED((NP, h4), jnp.float32),
            pltpu.SemaphoreType.DMA,
            pltpu.SemaphoreType.DMA,
        ],
    )
    return run(*zq, el, er, srcp, dstp)


def _tables(parts, bparts, swparts, dwparts, sxyb, dxyb):
    nparts = len(parts)
    dps = [p.shape[1] for p in parts]
    blk = 1024

    def body(*refs):
        part_refs = refs[:nparts]
        b_refs = refs[nparts:2 * nparts]
        sw_refs = refs[2 * nparts:3 * nparts]
        dw_refs = refs[3 * nparts:4 * nparts]
        sxy_ref = refs[4 * nparts]
        dxy_ref = refs[4 * nparts + 1]
        st_ref = refs[4 * nparts + 2]
        dt_ref = refs[4 * nparts + 3]
        st = sxy_ref[...]
        dt = dxy_ref[...]
        for p_ref, b_ref, sw_ref, dw_ref in zip(part_refs, b_refs, sw_refs,
                                                dw_refs):
            a = jnp.maximum(p_ref[...] + b_ref[...], 0.0)
            st = st + jnp.dot(a, sw_ref[...], preferred_element_type=jnp.float32)
            dt = dt + jnp.dot(a, dw_ref[...], preferred_element_type=jnp.float32)
        st_ref[...] = st
        dt_ref[...] = dt

    in_specs = (
        [pl.BlockSpec((blk, dp), lambda i: (i, 0)) for dp in dps]
        + [pl.BlockSpec((1, dp), lambda i: (0, 0)) for dp in dps]
        + [pl.BlockSpec((dp, 16), lambda i: (0, 0)) for dp in dps] * 2
        + [pl.BlockSpec((blk, 16), lambda i: (i, 0))] * 2
    )
    return pl.pallas_call(
        body,
        grid=(NP // blk,),
        in_specs=in_specs,
        out_specs=[pl.BlockSpec((blk, 16), lambda i: (i, 0))] * 2,
        out_shape=[jax.ShapeDtypeStruct((NP, 16), jnp.float32)] * 2,
    )(*parts, *bparts, *swparts, *dwparts, sxyb, dxyb)


def _sc_gather_tables(st, dt, srcp, dstp):
    def body(st_hbm, dt_hbm, src_hbm, dst_hbm, fa_hbm, fb_hbm,
             src_v, dst_v, rows_a0, rows_a1, rows_b0, rows_b1,
             sa0, sa1, sb0, sb1):
        c = lax.axis_index("c")
        s = lax.axis_index("s")
        base = (c * 16 + s) * ETG
        pltpu.sync_copy(src_hbm.at[pl.ds(base, ETG)], src_v)
        pltpu.sync_copy(dst_hbm.at[pl.ds(base, ETG)], dst_v)

        abufs = (rows_a0, rows_a1)
        bbufs = (rows_b0, rows_b1)
        asems = (sa0, sa1)
        bsems = (sb0, sb1)
        for b in range(2):
            pltpu.async_copy(st_hbm.at[src_v.at[pl.ds(b * 128, 128)]],
                             abufs[b], asems[b])
            pltpu.async_copy(dt_hbm.at[dst_v.at[pl.ds(b * 128, 128)]],
                             bbufs[b], bsems[b])

        def outer(j2, _):
            for b in range(2):
                j = j2 * 2 + b
                pltpu.make_async_copy(
                    st_hbm.at[src_v.at[pl.ds(j * 128, 128)]], abufs[b],
                    asems[b]).wait()
                pltpu.sync_copy(abufs[b],
                                fa_hbm.at[pl.ds(base + j * 128, 128)])
                pltpu.make_async_copy(
                    dt_hbm.at[dst_v.at[pl.ds(j * 128, 128)]], bbufs[b],
                    bsems[b]).wait()
                pltpu.sync_copy(bbufs[b],
                                fb_hbm.at[pl.ds(base + j * 128, 128)])

                @pl.when(j + 2 < ETG // 128)
                def _():
                    pltpu.async_copy(
                        st_hbm.at[src_v.at[pl.ds((j + 2) * 128, 128)]],
                        abufs[b], asems[b])
                    pltpu.async_copy(
                        dt_hbm.at[dst_v.at[pl.ds((j + 2) * 128, 128)]],
                        bbufs[b], bsems[b])
            return 0
        lax.fori_loop(0, ETG // 128 // 2, outer, 0)

    run = pl.kernel(
        body,
        compiler_params=_SC_PARAMS,
        out_type=(jax.ShapeDtypeStruct((EP, 16), jnp.float32),
                  jax.ShapeDtypeStruct((EP, 16), jnp.float32)),
        mesh=_MESH,
        scratch_types=[
            pltpu.VMEM((ETG,), jnp.int32),
            pltpu.VMEM((ETG,), jnp.int32),
            pltpu.VMEM((128, 16), jnp.float32),
            pltpu.VMEM((128, 16), jnp.float32),
            pltpu.VMEM((128, 16), jnp.float32),
            pltpu.VMEM((128, 16), jnp.float32),
            pltpu.SemaphoreType.DMA,
            pltpu.SemaphoreType.DMA,
            pltpu.SemaphoreType.DMA,
            pltpu.SemaphoreType.DMA,
        ],
    )
    return run(st, dt, srcp, dstp)


def _mlp(fa, fb, rap, w1a, w1b, b1, p1a, p1b, p2, w2, b2):
    blk = 8192

    def body(fa_ref, fb_ref, ra_ref, w1a_ref, w1b_ref, b1_ref, p1a_ref,
             p1b_ref, p2_ref, w2_ref, b2_ref, pc_ref, md_ref):
        ha = _prelu(fa_ref[...], p1a_ref[...])
        hb = _prelu(fb_ref[...], p1b_ref[...])
        h = (jnp.dot(ha, w1a_ref[...], preferred_element_type=jnp.float32)
             + jnp.dot(hb, w1b_ref[...], preferred_element_type=jnp.float32)
             + b1_ref[...])
        h = _prelu(h, p2_ref[...])
        pc = jnp.dot(h, w2_ref[...], preferred_element_type=jnp.float32) + b2_ref[...]
        pc_ref[...] = pc
        md_ref[...] = ra_ref[...] * (2.0 * jax.nn.sigmoid(pc) - 1.0)

    return pl.pallas_call(
        body,
        grid=(EP // blk,),
        in_specs=[
            pl.BlockSpec((blk, 16), lambda i: (i, 0)),
            pl.BlockSpec((blk, 16), lambda i: (i, 0)),
            pl.BlockSpec((blk, 1), lambda i: (i, 0)),
            pl.BlockSpec((16, 8), lambda i: (0, 0)),
            pl.BlockSpec((16, 8), lambda i: (0, 0)),
            pl.BlockSpec((1, 8), lambda i: (0, 0)),
            pl.BlockSpec((1, 16), lambda i: (0, 0)),
            pl.BlockSpec((1, 16), lambda i: (0, 0)),
            pl.BlockSpec((1, 8), lambda i: (0, 0)),
            pl.BlockSpec((8, 1), lambda i: (0, 0)),
            pl.BlockSpec((1, 1), lambda i: (0, 0)),
        ],
        out_specs=[
            pl.BlockSpec((blk, 1), lambda i: (i, 0)),
            pl.BlockSpec((blk, 1), lambda i: (i, 0)),
        ],
        out_shape=[
            jax.ShapeDtypeStruct((EP, 1), jnp.float32),
            jax.ShapeDtypeStruct((EP, 1), jnp.float32),
        ],
    )(fa, fb, rap, w1a, w1b, b1, p1a, p1b, p2, w2, b2)


def _sc_segmean_src(md, srcp):
    def body(md_hbm, src_hbm, out_hbm,
             msg_v, src_v, den_v, deg_v, idx_v, den_sh, deg_sh):
        c = lax.axis_index("c")
        s = lax.axis_index("s")

        @pl.when(c == 0)
        def _():
            base = s * EPT
            zeros16 = jnp.zeros((16,), jnp.float32)
            iota16 = lax.iota(jnp.int32, 16)
            pltpu.sync_copy(md_hbm.at[pl.ds(base, EPT)], msg_v)
            pltpu.sync_copy(src_hbm.at[pl.ds(base, EPT)], src_v)

            def zero(i, _):
                den_v[pl.ds(i * 16, 16)] = zeros16
                deg_v[pl.ds(i * 16, 16)] = zeros16
                return 0
            lax.fori_loop(0, NP // 16, zero, 0)
            pltpu.sync_copy(den_v.at[pl.ds(0, NROWS_T)],
                            den_sh.at[pl.ds(s * NROWS_T, NROWS_T)])
            pltpu.sync_copy(deg_v.at[pl.ds(0, NROWS_T)],
                            deg_sh.at[pl.ds(s * NROWS_T, NROWS_T)])

            def p(i, _):
                si = src_v[pl.ds(i * 16, 16)]
                mdv = msg_v[pl.ds(i * 16, 16)]
                gid = base + i * 16 + iota16
                w = jnp.where(gid < E, 1.0, 0.0)
                plsc.addupdate_scatter(den_v, [si], mdv)
                plsc.addupdate_scatter(deg_v, [si], w)
                return 0
            lax.fori_loop(0, EPT // 16, p, 0)
            plsc.subcore_barrier()

            def dred(j, _):
                for k in range(8):
                    idx_v[0, pl.ds(k * 16, 16)] = j * 128 + k * 16 + iota16
                pltpu.sync_copy(den_v.at[pl.ds(j * 128, 128)],
                                den_sh.at[idx_v.at[0]], add=True)
                pltpu.sync_copy(deg_v.at[pl.ds(j * 128, 128)],
                                deg_sh.at[idx_v.at[0]], add=True)
                return 0
            lax.fori_loop(0, NP // 128, dred, 0)
            plsc.subcore_barrier()

            pltpu.sync_copy(den_sh.at[pl.ds(s * NROWS_T, NROWS_T)],
                            den_v.at[pl.ds(0, NROWS_T)])
            pltpu.sync_copy(deg_sh.at[pl.ds(s * NROWS_T, NROWS_T)],
                            deg_v.at[pl.ds(0, NROWS_T)])

            def fin(i, _):
                d = den_v[pl.ds(i * 16, 16)]
                g = deg_v[pl.ds(i * 16, 16)]
                den_v[pl.ds(i * 16, 16)] = d / jnp.maximum(g, 1.0)
                return 0
            lax.fori_loop(0, NROWS_T // 16, fin, 0)
            pltpu.sync_copy(den_v.at[pl.ds(0, NROWS_T)],
                            out_hbm.at[pl.ds(s * NROWS_T, NROWS_T)])

    run = pl.kernel(
        body,
        compiler_params=_SC_PARAMS,
        out_type=jax.ShapeDtypeStruct((NP,), jnp.float32),
        mesh=_MESH,
        scratch_types=[
            pltpu.VMEM((EPT,), jnp.float32),
            pltpu.VMEM((EPT,), jnp.int32),
            pltpu.VMEM((NP,), jnp.float32),
            pltpu.VMEM((NP,), jnp.float32),
            pltpu.VMEM((1, 128), jnp.int32),
            pltpu.VMEM_SHARED((NP,), jnp.float32),
            pltpu.VMEM_SHARED((NP,), jnp.float32),
        ],
    )
    return run(md, srcp)


def kernel(features, cluster_features, xws, yws, raw_affine, edge_index, params):
    src = edge_index[0]
    dst = edge_index[1]
    srcp = jnp.pad(src, (0, EP - E))
    dstp = jnp.pad(dst, (0, EP - E))

    parts = [jnp.pad(features, ((0, NP - N), (0, 0))),
             jnp.pad(cluster_features, ((0, NP - N), (0, 0)))]
    bparts = [jnp.zeros((1, 128), jnp.float32)] * 2
    relu_in = False

    for i in range(4):
        w = params['conv%d_W' % i]
        b = params['conv%d_b' % i]
        a_s = params['conv%d_asrc' % i][:, None]
        a_d = params['conv%d_adst' % i][:, None]
        npart = len(parts)
        dp = w.shape[0] // npart
        h4 = w.shape[1] // 4
        wparts = [w[k * dp:(k + 1) * dp] for k in range(npart)]
        *zq, el, er = _gat_linear(parts, wparts, bparts, a_s, a_d, relu_in)
        aggq = _sc_gat_edge(zq, el.reshape(NP), er.reshape(NP), srcp, dstp)
        parts = list(aggq)
        bparts = [b[k * h4:(k + 1) * h4][None, :] for k in range(4)]
        relu_in = True

    xy = jnp.zeros((NP, 16), jnp.float32)
    xy = xy.at[:N, 6].set(xws[:, 0]).at[:N, 7].set(yws[:, 0])
    sxyb = xy.at[:, 0:6].add(params['src_b'][None, :])
    dxyb = xy.at[:, 0:6].add(params['dst_b'][None, :])
    swp = jnp.zeros((128, 16), jnp.float32).at[:, 0:6].set(params['src_W'])
    dwp = jnp.zeros((128, 16), jnp.float32).at[:, 0:6].set(params['dst_W'])
    swparts = [swp[k * 32:(k + 1) * 32] for k in range(4)]
    dwparts = [dwp[k * 32:(k + 1) * 32] for k in range(4)]
    st, dt = _tables(parts, bparts, swparts, dwparts, sxyb, dxyb)

    fa, fb = _sc_gather_tables(st, dt, srcp, dstp)

    rap = jnp.pad(raw_affine, (0, EP - E))[:, None]
    w1 = params['cls1_W']
    w1a = jnp.zeros((16, 8), jnp.float32).at[0:8].set(w1[0:8])
    w1b = jnp.zeros((16, 8), jnp.float32).at[0:8].set(w1[8:16])
    p1 = params['prelu1']
    p1a = jnp.zeros((1, 16), jnp.float32).at[0, 0:8].set(p1[0:8])
    p1b = jnp.zeros((1, 16), jnp.float32).at[0, 0:8].set(p1[8:16])
    pcp, mdp = _mlp(fa, fb, rap, w1a, w1b, params['cls1_b'][None, :],
                    p1a, p1b, params['prelu2'][None, :],
                    params['cls2_W'], params['cls2_b'][None, :])

    pred_den = _sc_segmean_src(mdp.reshape(EP), srcp)
    return pcp[:E], pred_den[:N]


# speedup vs baseline: 11.8587x; 1.0225x over previous
import jax
import jax.numpy as jnp
from jax import lax
from jax.experimental import pallas as pl
from jax.experimental.pallas import tpu as pltpu
from jax.experimental.pallas import tpu_sc as plsc

N = 10000
NP = 10240
E = 160000
EP = 163840
EPT = EP // 16
ETG = EP // 32
NCHUNK = EPT // 128
NROWS_T = NP // 16

_MESH = plsc.VectorSubcoreMesh(
    core_axis_name="c", subcore_axis_name="s", num_cores=2, num_subcores=16)
_SC_PARAMS = pltpu.CompilerParams(needs_layout_passes=False,
                                  use_tc_tiling_on_sc=False)


def _prelu(v, a):
    return jnp.where(v >= 0, v, a * v)


def _gat_linear(parts, wparts, bparts, a_s, a_d, relu_in):
    nparts = len(parts)
    dps = [p.shape[1] for p in parts]
    dout = wparts[0].shape[1]
    h4 = dout // 4
    blk = 1024

    def body(*refs):
        part_refs = refs[:nparts]
        w_refs = refs[nparts:2 * nparts]
        b_refs = refs[2 * nparts:3 * nparts]
        as_ref = refs[3 * nparts]
        ad_ref = refs[3 * nparts + 1]
        zq_refs = refs[3 * nparts + 2:3 * nparts + 6]
        el_ref = refs[3 * nparts + 6]
        er_ref = refs[3 * nparts + 7]
        z = None
        for p_ref, w_ref, b_ref in zip(part_refs, w_refs, b_refs):
            a = p_ref[...]
            if relu_in:
                a = jnp.maximum(a + b_ref[...], 0.0)
            d = jnp.dot(a, w_ref[...], preferred_element_type=jnp.float32)
            z = d if z is None else z + d
        for q in range(4):
            zq_refs[q][...] = z[:, q * h4:(q + 1) * h4]
        el_ref[...] = jnp.dot(z, as_ref[...], preferred_element_type=jnp.float32)
        er_ref[...] = jnp.dot(z, ad_ref[...], preferred_element_type=jnp.float32)

    in_specs = (
        [pl.BlockSpec((blk, dp), lambda i: (i, 0)) for dp in dps]
        + [pl.BlockSpec((dp, dout), lambda i: (0, 0)) for dp in dps]
        + [pl.BlockSpec((1, dp), lambda i: (0, 0)) for dp in dps]
        + [pl.BlockSpec((dout, 1), lambda i: (0, 0))] * 2
    )
    out_specs = ([pl.BlockSpec((blk, h4), lambda i: (i, 0))] * 4
                 + [pl.BlockSpec((blk, 1), lambda i: (i, 0))] * 2)
    out_shape = ([jax.ShapeDtypeStruct((NP, h4), jnp.float32)] * 4
                 + [jax.ShapeDtypeStruct((NP, 1), jnp.float32)] * 2)
    return pl.pallas_call(
        body,
        grid=(NP // blk,),
        in_specs=in_specs,
        out_specs=out_specs,
        out_shape=out_shape,
    )(*parts, *wparts, *bparts, a_s, a_d)


def _sc_gat_edge(zq, el, er, srcp, dstp):
    h4 = zq[0].shape[1]

    def body(z0_hbm, z1_hbm, z2_hbm, z3_hbm, el_hbm, er_hbm, src_hbm, dst_hbm,
             a0_hbm, a1_hbm, a2_hbm, a3_hbm,
             el_v, er_v, src_v, dst_v, alpha_v,
             rows0_v, rows1_v, rows2_v, rows3_v,
             idx0_v, idx1_v, idx2_v, idx3_v, stage_v, vmax_v,
             gmax_sh, denom_sh, acc_sh,
             gs0, gs1, gs2, gs3, ss0, ss1, ss2, ss3):
        c = lax.axis_index("c")
        s = lax.axis_index("s")
        base = s * EPT
        zeros16 = jnp.zeros((16,), jnp.float32)
        iota16 = lax.iota(jnp.int32, 16)
        denom_v = el_v

        pltpu.sync_copy(el_hbm, el_v)
        pltpu.sync_copy(er_hbm, er_v)
        pltpu.sync_copy(src_hbm.at[pl.ds(base, EPT)], src_v)
        pltpu.sync_copy(dst_hbm.at[pl.ds(base, EPT)], dst_v)

        def p1(i, vmax):
            si = src_v[pl.ds(i * 16, 16)]
            di = dst_v[pl.ds(i * 16, 16)]
            e = plsc.load_gather(el_v, [si]) + plsc.load_gather(er_v, [di])
            e = jnp.where(e >= 0, e, 0.2 * e)
            alpha_v[pl.ds(i * 16, 16)] = e
            return jnp.maximum(vmax, e)
        vmax = lax.fori_loop(0, EPT // 16, p1,
                             jnp.full((16,), -3.0e38, jnp.float32))
        vmax_v[...] = vmax
        pltpu.sync_copy(vmax_v, gmax_sh.at[s])
        plsc.subcore_barrier()
        pltpu.sync_copy(gmax_sh, stage_v)
        m = stage_v[0]
        for t in range(1, 16):
            m = jnp.maximum(m, stage_v[t])
        gmax = jnp.max(m)

        def zden(i, _):
            denom_v[pl.ds(i * 16, 16)] = zeros16
            return 0
        lax.fori_loop(0, NP // 16, zden, 0)
        pltpu.sync_copy(denom_v.at[pl.ds(0, NROWS_T)],
                        denom_sh.at[pl.ds(s * NROWS_T, NROWS_T)])

        def p2(i, _):
            di = dst_v[pl.ds(i * 16, 16)]
            e = alpha_v[pl.ds(i * 16, 16)]
            gid = base + i * 16 + iota16
            ex = jnp.exp(e - gmax)
            ex = jnp.where(gid < E, ex, 0.0)
            alpha_v[pl.ds(i * 16, 16)] = ex
            plsc.addupdate_scatter(denom_v, [di], ex)
            return 0
        lax.fori_loop(0, EPT // 16, p2, 0)
        plsc.subcore_barrier()

        def dred(j, _):
            for k in range(8):
                idx0_v[0, pl.ds(k * 16, 16)] = j * 128 + k * 16 + iota16
            pltpu.sync_copy(denom_v.at[pl.ds(j * 128, 128)],
                            denom_sh.at[idx0_v.at[0]], add=True)
            return 0
        lax.fori_loop(0, NP // 128, dred, 0)
        plsc.subcore_barrier()
        pltpu.sync_copy(denom_sh, denom_v)

        def p3(i, _):
            di = dst_v[pl.ds(i * 16, 16)]
            den = plsc.load_gather(denom_v, [di])
            alpha_v[pl.ds(i * 16, 16)] = (
                alpha_v[pl.ds(i * 16, 16)] / jnp.maximum(den, 1e-30))
            return 0
        lax.fori_loop(0, EPT // 16, p3, 0)

        bufs = (rows0_v, rows1_v, rows2_v, rows3_v)
        idxs = (idx0_v, idx1_v, idx2_v, idx3_v)
        gsems = (gs0, gs1, gs2, gs3)
        ssems = (ss0, ss1, ss2, ss3)

        def phase_b(z_hbm, agg_hbm):
            def zrow(i, _):
                for k in range(h4 // 16):
                    rows0_v[i, pl.ds(k * 16, 16)] = zeros16
                return 0
            lax.fori_loop(0, 128, zrow, 0)

            def zacc(j, _):
                pltpu.sync_copy(rows0_v,
                                acc_sh.at[pl.ds(s * NROWS_T + j * 128, 128)])
                return 0
            lax.fori_loop(0, NROWS_T // 128, zacc, 0)
            plsc.subcore_barrier()

            for b in range(2):
                pltpu.async_copy(z_hbm.at[src_v.at[pl.ds(b * 128, 128)]],
                                 bufs[b], gsems[b])

            def outer(j4, _):
                for b in range(4):
                    j = j4 * 4 + b
                    bg = (b + 2) % 4
                    pltpu.make_async_copy(
                        z_hbm.at[src_v.at[pl.ds(j * 128, 128)]], bufs[b],
                        gsems[b]).wait()

                    def grp(g, _):
                        av = alpha_v[pl.ds(j * 128 + g * 16, 16)]
                        for r in range(16):
                            arow = jnp.full((16,), av[r], jnp.float32)
                            row = g * 16 + r
                            for k in range(h4 // 16):
                                bufs[b][row, pl.ds(k * 16, 16)] = (
                                    bufs[b][row, pl.ds(k * 16, 16)] * arow)
                        return 0
                    lax.fori_loop(0, 8, grp, 0)
                    for k in range(8):
                        idxs[b][0, pl.ds(k * 16, 16)] = (
                            dst_v[pl.ds(j * 128 + k * 16, 16)])
                    pltpu.async_copy(bufs[b], acc_sh.at[idxs[b].at[0]],
                                     ssems[b], add=True)

                    @pl.when(j >= 2)
                    def _():
                        pltpu.make_async_copy(
                            bufs[bg], acc_sh.at[idxs[bg].at[0]],
                            ssems[bg]).wait()

                    @pl.when(j + 2 < NCHUNK)
                    def _():
                        pltpu.async_copy(
                            z_hbm.at[src_v.at[pl.ds((j + 2) * 128, 128)]],
                            bufs[bg], gsems[bg])
                return 0
            lax.fori_loop(0, NCHUNK // 4, outer, 0)
            for j in (NCHUNK - 2, NCHUNK - 1):
                b = j % 4
                pltpu.make_async_copy(bufs[b], acc_sh.at[idxs[b].at[0]],
                                      ssems[b]).wait()
            plsc.subcore_barrier()
            pltpu.sync_copy(acc_sh.at[pl.ds(s * NROWS_T, NROWS_T)],
                            agg_hbm.at[pl.ds(s * NROWS_T, NROWS_T)])

        @pl.when(c == 0)
        def _():
            phase_b(z0_hbm, a0_hbm)
            phase_b(z1_hbm, a1_hbm)

        @pl.when(c == 1)
        def _():
            phase_b(z2_hbm, a2_hbm)
            phase_b(z3_hbm, a3_hbm)

    run = pl.kernel(
        body,
        compiler_params=_SC_PARAMS,
        out_type=tuple(jax.ShapeDtypeStruct((NP, h4), jnp.float32)
                       for _ in range(4)),
        mesh=_MESH,
        scratch_types=[
            pltpu.VMEM((NP,), jnp.float32),
            pltpu.VMEM((NP,), jnp.float32),
            pltpu.VMEM((EPT,), jnp.int32),
            pltpu.VMEM((EPT,), jnp.int32),
            pltpu.VMEM((EPT,), jnp.float32),
            pltpu.VMEM((128, h4), jnp.float32),
            pltpu.VMEM((128, h4), jnp.float32),
            pltpu.VMEM((128, h4), jnp.float32),
            pltpu.VMEM((128, h4), jnp.float32),
            pltpu.VMEM((1, 128), jnp.int32),
            pltpu.VMEM((1, 128), jnp.int32),
            pltpu.VMEM((1, 128), jnp.int32),
            pltpu.VMEM((1, 128), jnp.int32),
            pltpu.VMEM((16, 16), jnp.float32),
            pltpu.VMEM((16,), jnp.float32),
            pltpu.VMEM_SHARED((16, 16), jnp.float32),
            pltpu.VMEM_SHARED((NP,), jnp.float32),
            pltpu.VMEM_SHARED((NP, h4), jnp.float32),
        ] + [pltpu.SemaphoreType.DMA] * 8,
    )
    return run(*zq, el, er, srcp, dstp)


def _tables(parts, bparts, swparts, dwparts, sxyb, dxyb):
    nparts = len(parts)
    dps = [p.shape[1] for p in parts]
    blk = 1024

    def body(*refs):
        part_refs = refs[:nparts]
        b_refs = refs[nparts:2 * nparts]
        sw_refs = refs[2 * nparts:3 * nparts]
        dw_refs = refs[3 * nparts:4 * nparts]
        sxy_ref = refs[4 * nparts]
        dxy_ref = refs[4 * nparts + 1]
        st_ref = refs[4 * nparts + 2]
        dt_ref = refs[4 * nparts + 3]
        st = sxy_ref[...]
        dt = dxy_ref[...]
        for p_ref, b_ref, sw_ref, dw_ref in zip(part_refs, b_refs, sw_refs,
                                                dw_refs):
            a = jnp.maximum(p_ref[...] + b_ref[...], 0.0)
            st = st + jnp.dot(a, sw_ref[...], preferred_element_type=jnp.float32)
            dt = dt + jnp.dot(a, dw_ref[...], preferred_element_type=jnp.float32)
        st_ref[...] = st
        dt_ref[...] = dt

    in_specs = (
        [pl.BlockSpec((blk, dp), lambda i: (i, 0)) for dp in dps]
        + [pl.BlockSpec((1, dp), lambda i: (0, 0)) for dp in dps]
        + [pl.BlockSpec((dp, 16), lambda i: (0, 0)) for dp in dps] * 2
        + [pl.BlockSpec((blk, 16), lambda i: (i, 0))] * 2
    )
    return pl.pallas_call(
        body,
        grid=(NP // blk,),
        in_specs=in_specs,
        out_specs=[pl.BlockSpec((blk, 16), lambda i: (i, 0))] * 2,
        out_shape=[jax.ShapeDtypeStruct((NP, 16), jnp.float32)] * 2,
    )(*parts, *bparts, *swparts, *dwparts, sxyb, dxyb)


def _sc_gather_tables(st, dt, srcp, dstp):
    def body(st_hbm, dt_hbm, src_hbm, dst_hbm, fa_hbm, fb_hbm,
             src_v, dst_v, rows_a0, rows_a1, rows_b0, rows_b1,
             sa0, sa1, sb0, sb1):
        c = lax.axis_index("c")
        s = lax.axis_index("s")
        base = (c * 16 + s) * ETG
        pltpu.sync_copy(src_hbm.at[pl.ds(base, ETG)], src_v)
        pltpu.sync_copy(dst_hbm.at[pl.ds(base, ETG)], dst_v)

        abufs = (rows_a0, rows_a1)
        bbufs = (rows_b0, rows_b1)
        asems = (sa0, sa1)
        bsems = (sb0, sb1)
        for b in range(2):
            pltpu.async_copy(st_hbm.at[src_v.at[pl.ds(b * 128, 128)]],
                             abufs[b], asems[b])
            pltpu.async_copy(dt_hbm.at[dst_v.at[pl.ds(b * 128, 128)]],
                             bbufs[b], bsems[b])

        def outer(j2, _):
            for b in range(2):
                j = j2 * 2 + b
                pltpu.make_async_copy(
                    st_hbm.at[src_v.at[pl.ds(j * 128, 128)]], abufs[b],
                    asems[b]).wait()
                pltpu.sync_copy(abufs[b],
                                fa_hbm.at[pl.ds(base + j * 128, 128)])
                pltpu.make_async_copy(
                    dt_hbm.at[dst_v.at[pl.ds(j * 128, 128)]], bbufs[b],
                    bsems[b]).wait()
                pltpu.sync_copy(bbufs[b],
                                fb_hbm.at[pl.ds(base + j * 128, 128)])

                @pl.when(j + 2 < ETG // 128)
                def _():
                    pltpu.async_copy(
                        st_hbm.at[src_v.at[pl.ds((j + 2) * 128, 128)]],
                        abufs[b], asems[b])
                    pltpu.async_copy(
                        dt_hbm.at[dst_v.at[pl.ds((j + 2) * 128, 128)]],
                        bbufs[b], bsems[b])
            return 0
        lax.fori_loop(0, ETG // 128 // 2, outer, 0)

    run = pl.kernel(
        body,
        compiler_params=_SC_PARAMS,
        out_type=(jax.ShapeDtypeStruct((EP, 16), jnp.float32),
                  jax.ShapeDtypeStruct((EP, 16), jnp.float32)),
        mesh=_MESH,
        scratch_types=[
            pltpu.VMEM((ETG,), jnp.int32),
            pltpu.VMEM((ETG,), jnp.int32),
            pltpu.VMEM((128, 16), jnp.float32),
            pltpu.VMEM((128, 16), jnp.float32),
            pltpu.VMEM((128, 16), jnp.float32),
            pltpu.VMEM((128, 16), jnp.float32),
            pltpu.SemaphoreType.DMA,
            pltpu.SemaphoreType.DMA,
            pltpu.SemaphoreType.DMA,
            pltpu.SemaphoreType.DMA,
        ],
    )
    return run(st, dt, srcp, dstp)


def _mlp(fa, fb, rap, w1a, w1b, b1, p1a, p1b, p2, w2, b2):
    blk = 8192

    def body(fa_ref, fb_ref, ra_ref, w1a_ref, w1b_ref, b1_ref, p1a_ref,
             p1b_ref, p2_ref, w2_ref, b2_ref, pc_ref, md_ref):
        ha = _prelu(fa_ref[...], p1a_ref[...])
        hb = _prelu(fb_ref[...], p1b_ref[...])
        h = (jnp.dot(ha, w1a_ref[...], preferred_element_type=jnp.float32)
             + jnp.dot(hb, w1b_ref[...], preferred_element_type=jnp.float32)
             + b1_ref[...])
        h = _prelu(h, p2_ref[...])
        pc = jnp.dot(h, w2_ref[...], preferred_element_type=jnp.float32) + b2_ref[...]
        pc_ref[...] = pc
        md_ref[...] = ra_ref[...] * (2.0 * jax.nn.sigmoid(pc) - 1.0)

    return pl.pallas_call(
        body,
        grid=(EP // blk,),
        in_specs=[
            pl.BlockSpec((blk, 16), lambda i: (i, 0)),
            pl.BlockSpec((blk, 16), lambda i: (i, 0)),
            pl.BlockSpec((blk, 1), lambda i: (i, 0)),
            pl.BlockSpec((16, 8), lambda i: (0, 0)),
            pl.BlockSpec((16, 8), lambda i: (0, 0)),
            pl.BlockSpec((1, 8), lambda i: (0, 0)),
            pl.BlockSpec((1, 16), lambda i: (0, 0)),
            pl.BlockSpec((1, 16), lambda i: (0, 0)),
            pl.BlockSpec((1, 8), lambda i: (0, 0)),
            pl.BlockSpec((8, 1), lambda i: (0, 0)),
            pl.BlockSpec((1, 1), lambda i: (0, 0)),
        ],
        out_specs=[
            pl.BlockSpec((blk, 1), lambda i: (i, 0)),
            pl.BlockSpec((blk, 1), lambda i: (i, 0)),
        ],
        out_shape=[
            jax.ShapeDtypeStruct((EP, 1), jnp.float32),
            jax.ShapeDtypeStruct((EP, 1), jnp.float32),
        ],
    )(fa, fb, rap, w1a, w1b, b1, p1a, p1b, p2, w2, b2)


def _sc_segmean_src(md, srcp):
    def body(md_hbm, src_hbm, out_hbm,
             msg_v, src_v, den_v, deg_v, idx_v, den_sh, deg_sh):
        c = lax.axis_index("c")
        s = lax.axis_index("s")

        @pl.when(c == 0)
        def _():
            base = s * EPT
            zeros16 = jnp.zeros((16,), jnp.float32)
            iota16 = lax.iota(jnp.int32, 16)
            pltpu.sync_copy(md_hbm.at[pl.ds(base, EPT)], msg_v)
            pltpu.sync_copy(src_hbm.at[pl.ds(base, EPT)], src_v)

            def zero(i, _):
                den_v[pl.ds(i * 16, 16)] = zeros16
                deg_v[pl.ds(i * 16, 16)] = zeros16
                return 0
            lax.fori_loop(0, NP // 16, zero, 0)
            pltpu.sync_copy(den_v.at[pl.ds(0, NROWS_T)],
                            den_sh.at[pl.ds(s * NROWS_T, NROWS_T)])
            pltpu.sync_copy(deg_v.at[pl.ds(0, NROWS_T)],
                            deg_sh.at[pl.ds(s * NROWS_T, NROWS_T)])

            def p(i, _):
                si = src_v[pl.ds(i * 16, 16)]
                mdv = msg_v[pl.ds(i * 16, 16)]
                gid = base + i * 16 + iota16
                w = jnp.where(gid < E, 1.0, 0.0)
                plsc.addupdate_scatter(den_v, [si], mdv)
                plsc.addupdate_scatter(deg_v, [si], w)
                return 0
            lax.fori_loop(0, EPT // 16, p, 0)
            plsc.subcore_barrier()

            def dred(j, _):
                for k in range(8):
                    idx_v[0, pl.ds(k * 16, 16)] = j * 128 + k * 16 + iota16
                pltpu.sync_copy(den_v.at[pl.ds(j * 128, 128)],
                                den_sh.at[idx_v.at[0]], add=True)
                pltpu.sync_copy(deg_v.at[pl.ds(j * 128, 128)],
                                deg_sh.at[idx_v.at[0]], add=True)
                return 0
            lax.fori_loop(0, NP // 128, dred, 0)
            plsc.subcore_barrier()

            pltpu.sync_copy(den_sh.at[pl.ds(s * NROWS_T, NROWS_T)],
                            den_v.at[pl.ds(0, NROWS_T)])
            pltpu.sync_copy(deg_sh.at[pl.ds(s * NROWS_T, NROWS_T)],
                            deg_v.at[pl.ds(0, NROWS_T)])

            def fin(i, _):
                d = den_v[pl.ds(i * 16, 16)]
                g = deg_v[pl.ds(i * 16, 16)]
                den_v[pl.ds(i * 16, 16)] = d / jnp.maximum(g, 1.0)
                return 0
            lax.fori_loop(0, NROWS_T // 16, fin, 0)
            pltpu.sync_copy(den_v.at[pl.ds(0, NROWS_T)],
                            out_hbm.at[pl.ds(s * NROWS_T, NROWS_T)])

    run = pl.kernel(
        body,
        compiler_params=_SC_PARAMS,
        out_type=jax.ShapeDtypeStruct((NP,), jnp.float32),
        mesh=_MESH,
        scratch_types=[
            pltpu.VMEM((EPT,), jnp.float32),
            pltpu.VMEM((EPT,), jnp.int32),
            pltpu.VMEM((NP,), jnp.float32),
            pltpu.VMEM((NP,), jnp.float32),
            pltpu.VMEM((1, 128), jnp.int32),
            pltpu.VMEM_SHARED((NP,), jnp.float32),
            pltpu.VMEM_SHARED((NP,), jnp.float32),
        ],
    )
    return run(md, srcp)


def kernel(features, cluster_features, xws, yws, raw_affine, edge_index, params):
    src = edge_index[0]
    dst = edge_index[1]
    srcp = jnp.pad(src, (0, EP - E))
    dstp = jnp.pad(dst, (0, EP - E))

    parts = [jnp.pad(features, ((0, NP - N), (0, 0))),
             jnp.pad(cluster_features, ((0, NP - N), (0, 0)))]
    bparts = [jnp.zeros((1, 128), jnp.float32)] * 2
    relu_in = False

    for i in range(4):
        w = params['conv%d_W' % i]
        b = params['conv%d_b' % i]
        a_s = params['conv%d_asrc' % i][:, None]
        a_d = params['conv%d_adst' % i][:, None]
        npart = len(parts)
        dp = w.shape[0] // npart
        h4 = w.shape[1] // 4
        wparts = [w[k * dp:(k + 1) * dp] for k in range(npart)]
        *zq, el, er = _gat_linear(parts, wparts, bparts, a_s, a_d, relu_in)
        aggq = _sc_gat_edge(zq, el.reshape(NP), er.reshape(NP), srcp, dstp)
        parts = list(aggq)
        bparts = [b[k * h4:(k + 1) * h4][None, :] for k in range(4)]
        relu_in = True

    xy = jnp.zeros((NP, 16), jnp.float32)
    xy = xy.at[:N, 6].set(xws[:, 0]).at[:N, 7].set(yws[:, 0])
    sxyb = xy.at[:, 0:6].add(params['src_b'][None, :])
    dxyb = xy.at[:, 0:6].add(params['dst_b'][None, :])
    swp = jnp.zeros((128, 16), jnp.float32).at[:, 0:6].set(params['src_W'])
    dwp = jnp.zeros((128, 16), jnp.float32).at[:, 0:6].set(params['dst_W'])
    swparts = [swp[k * 32:(k + 1) * 32] for k in range(4)]
    dwparts = [dwp[k * 32:(k + 1) * 32] for k in range(4)]
    st, dt = _tables(parts, bparts, swparts, dwparts, sxyb, dxyb)

    fa, fb = _sc_gather_tables(st, dt, srcp, dstp)

    rap = jnp.pad(raw_affine, (0, EP - E))[:, None]
    w1 = params['cls1_W']
    w1a = jnp.zeros((16, 8), jnp.float32).at[0:8].set(w1[0:8])
    w1b = jnp.zeros((16, 8), jnp.float32).at[0:8].set(w1[8:16])
    p1 = params['prelu1']
    p1a = jnp.zeros((1, 16), jnp.float32).at[0, 0:8].set(p1[0:8])
    p1b = jnp.zeros((1, 16), jnp.float32).at[0, 0:8].set(p1[8:16])
    pcp, mdp = _mlp(fa, fb, rap, w1a, w1b, params['cls1_b'][None, :],
                    p1a, p1b, params['prelu2'][None, :],
                    params['cls2_W'], params['cls2_b'][None, :])

    pred_den = _sc_segmean_src(mdp.reshape(EP), srcp)
    return pcp[:E], pred_den[:N]


# speedup vs baseline: 11.9613x; 1.0087x over previous
import jax
import jax.numpy as jnp
from jax import lax
from jax.experimental import pallas as pl
from jax.experimental.pallas import tpu as pltpu
from jax.experimental.pallas import tpu_sc as plsc

N = 10000
NP = 10240
E = 160000
EP = 163840
EPT = EP // 16
ETG = EP // 32
NCHUNK = EPT // 128
NROWS_T = NP // 16

_MESH = plsc.VectorSubcoreMesh(
    core_axis_name="c", subcore_axis_name="s", num_cores=2, num_subcores=16)
_SC_PARAMS = pltpu.CompilerParams(needs_layout_passes=False,
                                  use_tc_tiling_on_sc=False)


def _prelu(v, a):
    return jnp.where(v >= 0, v, a * v)


def _gat_linear(parts, wparts, bparts, a_s, a_d, relu_in):
    nparts = len(parts)
    dps = [p.shape[1] for p in parts]
    dout = wparts[0].shape[1]
    h4 = dout // 4
    blk = 1024

    def body(*refs):
        part_refs = refs[:nparts]
        w_refs = refs[nparts:2 * nparts]
        b_refs = refs[2 * nparts:3 * nparts]
        as_ref = refs[3 * nparts]
        ad_ref = refs[3 * nparts + 1]
        zq_refs = refs[3 * nparts + 2:3 * nparts + 6]
        el_ref = refs[3 * nparts + 6]
        er_ref = refs[3 * nparts + 7]
        z = None
        for p_ref, w_ref, b_ref in zip(part_refs, w_refs, b_refs):
            a = p_ref[...]
            if relu_in:
                a = jnp.maximum(a + b_ref[...], 0.0)
            d = jnp.dot(a, w_ref[...], preferred_element_type=jnp.float32)
            z = d if z is None else z + d
        for q in range(4):
            zq_refs[q][...] = z[:, q * h4:(q + 1) * h4]
        el_ref[...] = jnp.dot(z, as_ref[...], preferred_element_type=jnp.float32)
        er_ref[...] = jnp.dot(z, ad_ref[...], preferred_element_type=jnp.float32)

    in_specs = (
        [pl.BlockSpec((blk, dp), lambda i: (i, 0)) for dp in dps]
        + [pl.BlockSpec((dp, dout), lambda i: (0, 0)) for dp in dps]
        + [pl.BlockSpec((1, dp), lambda i: (0, 0)) for dp in dps]
        + [pl.BlockSpec((dout, 1), lambda i: (0, 0))] * 2
    )
    out_specs = ([pl.BlockSpec((blk, h4), lambda i: (i, 0))] * 4
                 + [pl.BlockSpec((blk, 1), lambda i: (i, 0))] * 2)
    out_shape = ([jax.ShapeDtypeStruct((NP, h4), jnp.float32)] * 4
                 + [jax.ShapeDtypeStruct((NP, 1), jnp.float32)] * 2)
    return pl.pallas_call(
        body,
        grid=(NP // blk,),
        in_specs=in_specs,
        out_specs=out_specs,
        out_shape=out_shape,
    )(*parts, *wparts, *bparts, a_s, a_d)


def _sc_gat_edge(zq, el, er, srcp, dstp):
    h4 = zq[0].shape[1]

    def body(z0_hbm, z1_hbm, z2_hbm, z3_hbm, el_hbm, er_hbm, src_hbm, dst_hbm,
             a0_hbm, a1_hbm, a2_hbm, a3_hbm,
             el_v, er_v, src_v, dst_v, alpha_v,
             rows0_v, rows1_v, rows2_v, rows3_v,
             idx0_v, idx1_v, idx2_v, idx3_v, stage_v, vmax_v,
             gmax_sh, denom_sh, acc_sh,
             gs0, gs1, gs2, gs3, ss0, ss1, ss2, ss3):
        c = lax.axis_index("c")
        s = lax.axis_index("s")
        base = s * EPT
        zeros16 = jnp.zeros((16,), jnp.float32)
        iota16 = lax.iota(jnp.int32, 16)
        denom_v = el_v

        pltpu.sync_copy(el_hbm, el_v)
        pltpu.sync_copy(er_hbm, er_v)
        pltpu.sync_copy(src_hbm.at[pl.ds(base, EPT)], src_v)
        pltpu.sync_copy(dst_hbm.at[pl.ds(base, EPT)], dst_v)

        def p1(i, vmax):
            si = src_v[pl.ds(i * 16, 16)]
            di = dst_v[pl.ds(i * 16, 16)]
            e = plsc.load_gather(el_v, [si]) + plsc.load_gather(er_v, [di])
            e = jnp.where(e >= 0, e, 0.2 * e)
            alpha_v[pl.ds(i * 16, 16)] = e
            return jnp.maximum(vmax, e)
        vmax = lax.fori_loop(0, EPT // 16, p1,
                             jnp.full((16,), -3.0e38, jnp.float32))
        vmax_v[...] = vmax
        pltpu.sync_copy(vmax_v, gmax_sh.at[s])
        plsc.subcore_barrier()
        pltpu.sync_copy(gmax_sh, stage_v)
        m = stage_v[0]
        for t in range(1, 16):
            m = jnp.maximum(m, stage_v[t])
        gmax = jnp.max(m)

        def zden(i, _):
            denom_v[pl.ds(i * 16, 16)] = zeros16
            return 0
        lax.fori_loop(0, NP // 16, zden, 0)
        pltpu.sync_copy(denom_v.at[pl.ds(0, NROWS_T)],
                        denom_sh.at[pl.ds(s * NROWS_T, NROWS_T)])

        def p2(i, _):
            di = dst_v[pl.ds(i * 16, 16)]
            e = alpha_v[pl.ds(i * 16, 16)]
            gid = base + i * 16 + iota16
            ex = jnp.exp(e - gmax)
            ex = jnp.where(gid < E, ex, 0.0)
            alpha_v[pl.ds(i * 16, 16)] = ex
            plsc.addupdate_scatter(denom_v, [di], ex)
            return 0
        lax.fori_loop(0, EPT // 16, p2, 0)
        plsc.subcore_barrier()

        idxb = (idx0_v, idx1_v, idx2_v, idx3_v)
        dsem = (gs0, gs1, gs2, gs3)

        def dred(j4, _):
            for b in range(4):
                j = j4 * 4 + b
                for k in range(8):
                    idxb[b][0, pl.ds(k * 16, 16)] = j * 128 + k * 16 + iota16
                pltpu.async_copy(denom_v.at[pl.ds(j * 128, 128)],
                                 denom_sh.at[idxb[b].at[0]], dsem[b],
                                 add=True)
            for b in range(4):
                j = j4 * 4 + b
                pltpu.make_async_copy(denom_v.at[pl.ds(j * 128, 128)],
                                      denom_sh.at[idxb[b].at[0]],
                                      dsem[b]).wait()
            return 0
        lax.fori_loop(0, NP // 128 // 4, dred, 0)
        plsc.subcore_barrier()
        pltpu.sync_copy(denom_sh, denom_v)

        def p3(i, _):
            di = dst_v[pl.ds(i * 16, 16)]
            den = plsc.load_gather(denom_v, [di])
            alpha_v[pl.ds(i * 16, 16)] = (
                alpha_v[pl.ds(i * 16, 16)] / jnp.maximum(den, 1e-30))
            return 0
        lax.fori_loop(0, EPT // 16, p3, 0)

        bufs = (rows0_v, rows1_v, rows2_v, rows3_v)
        idxs = (idx0_v, idx1_v, idx2_v, idx3_v)
        gsems = (gs0, gs1, gs2, gs3)
        ssems = (ss0, ss1, ss2, ss3)

        def phase_b(z_hbm, agg_hbm):
            def zrow(i, _):
                for k in range(h4 // 16):
                    rows0_v[i, pl.ds(k * 16, 16)] = zeros16
                return 0
            lax.fori_loop(0, 128, zrow, 0)

            def zacc(j, _):
                pltpu.sync_copy(rows0_v,
                                acc_sh.at[pl.ds(s * NROWS_T + j * 128, 128)])
                return 0
            lax.fori_loop(0, NROWS_T // 128, zacc, 0)
            plsc.subcore_barrier()

            for b in range(2):
                pltpu.async_copy(z_hbm.at[src_v.at[pl.ds(b * 128, 128)]],
                                 bufs[b], gsems[b])

            def outer(j4, _):
                for b in range(4):
                    j = j4 * 4 + b
                    bg = (b + 2) % 4
                    pltpu.make_async_copy(
                        z_hbm.at[src_v.at[pl.ds(j * 128, 128)]], bufs[b],
                        gsems[b]).wait()

                    def grp(g, _):
                        av = alpha_v[pl.ds(j * 128 + g * 16, 16)]
                        for r in range(16):
                            arow = jnp.full((16,), av[r], jnp.float32)
                            row = g * 16 + r
                            for k in range(h4 // 16):
                                bufs[b][row, pl.ds(k * 16, 16)] = (
                                    bufs[b][row, pl.ds(k * 16, 16)] * arow)
                        return 0
                    lax.fori_loop(0, 8, grp, 0)
                    for k in range(8):
                        idxs[b][0, pl.ds(k * 16, 16)] = (
                            dst_v[pl.ds(j * 128 + k * 16, 16)])
                    pltpu.async_copy(bufs[b], acc_sh.at[idxs[b].at[0]],
                                     ssems[b], add=True)

                    @pl.when(j >= 2)
                    def _():
                        pltpu.make_async_copy(
                            bufs[bg], acc_sh.at[idxs[bg].at[0]],
                            ssems[bg]).wait()

                    @pl.when(j + 2 < NCHUNK)
                    def _():
                        pltpu.async_copy(
                            z_hbm.at[src_v.at[pl.ds((j + 2) * 128, 128)]],
                            bufs[bg], gsems[bg])
                return 0
            lax.fori_loop(0, NCHUNK // 4, outer, 0)
            for j in (NCHUNK - 2, NCHUNK - 1):
                b = j % 4
                pltpu.make_async_copy(bufs[b], acc_sh.at[idxs[b].at[0]],
                                      ssems[b]).wait()
            plsc.subcore_barrier()
            pltpu.sync_copy(acc_sh.at[pl.ds(s * NROWS_T, NROWS_T)],
                            agg_hbm.at[pl.ds(s * NROWS_T, NROWS_T)])

        @pl.when(c == 0)
        def _():
            phase_b(z0_hbm, a0_hbm)
            phase_b(z1_hbm, a1_hbm)

        @pl.when(c == 1)
        def _():
            phase_b(z2_hbm, a2_hbm)
            phase_b(z3_hbm, a3_hbm)

    run = pl.kernel(
        body,
        compiler_params=_SC_PARAMS,
        out_type=tuple(jax.ShapeDtypeStruct((NP, h4), jnp.float32)
                       for _ in range(4)),
        mesh=_MESH,
        scratch_types=[
            pltpu.VMEM((NP,), jnp.float32),
            pltpu.VMEM((NP,), jnp.float32),
            pltpu.VMEM((EPT,), jnp.int32),
            pltpu.VMEM((EPT,), jnp.int32),
            pltpu.VMEM((EPT,), jnp.float32),
            pltpu.VMEM((128, h4), jnp.float32),
            pltpu.VMEM((128, h4), jnp.float32),
            pltpu.VMEM((128, h4), jnp.float32),
            pltpu.VMEM((128, h4), jnp.float32),
            pltpu.VMEM((1, 128), jnp.int32),
            pltpu.VMEM((1, 128), jnp.int32),
            pltpu.VMEM((1, 128), jnp.int32),
            pltpu.VMEM((1, 128), jnp.int32),
            pltpu.VMEM((16, 16), jnp.float32),
            pltpu.VMEM((16,), jnp.float32),
            pltpu.VMEM_SHARED((16, 16), jnp.float32),
            pltpu.VMEM_SHARED((NP,), jnp.float32),
            pltpu.VMEM_SHARED((NP, h4), jnp.float32),
        ] + [pltpu.SemaphoreType.DMA] * 8,
    )
    return run(*zq, el, er, srcp, dstp)


def _tables(parts, bparts, swparts, dwparts, sxyb, dxyb):
    nparts = len(parts)
    dps = [p.shape[1] for p in parts]
    blk = 1024

    def body(*refs):
        part_refs = refs[:nparts]
        b_refs = refs[nparts:2 * nparts]
        sw_refs = refs[2 * nparts:3 * nparts]
        dw_refs = refs[3 * nparts:4 * nparts]
        sxy_ref = refs[4 * nparts]
        dxy_ref = refs[4 * nparts + 1]
        st_ref = refs[4 * nparts + 2]
        dt_ref = refs[4 * nparts + 3]
        st = sxy_ref[...]
        dt = dxy_ref[...]
        for p_ref, b_ref, sw_ref, dw_ref in zip(part_refs, b_refs, sw_refs,
                                                dw_refs):
            a = jnp.maximum(p_ref[...] + b_ref[...], 0.0)
            st = st + jnp.dot(a, sw_ref[...], preferred_element_type=jnp.float32)
            dt = dt + jnp.dot(a, dw_ref[...], preferred_element_type=jnp.float32)
        st_ref[...] = st
        dt_ref[...] = dt

    in_specs = (
        [pl.BlockSpec((blk, dp), lambda i: (i, 0)) for dp in dps]
        + [pl.BlockSpec((1, dp), lambda i: (0, 0)) for dp in dps]
        + [pl.BlockSpec((dp, 16), lambda i: (0, 0)) for dp in dps] * 2
        + [pl.BlockSpec((blk, 16), lambda i: (i, 0))] * 2
    )
    return pl.pallas_call(
        body,
        grid=(NP // blk,),
        in_specs=in_specs,
        out_specs=[pl.BlockSpec((blk, 16), lambda i: (i, 0))] * 2,
        out_shape=[jax.ShapeDtypeStruct((NP, 16), jnp.float32)] * 2,
    )(*parts, *bparts, *swparts, *dwparts, sxyb, dxyb)


def _sc_gather_tables(st, dt, srcp, dstp):
    def body(st_hbm, dt_hbm, src_hbm, dst_hbm, fa_hbm, fb_hbm,
             src_v, dst_v, rows_a0, rows_a1, rows_b0, rows_b1,
             sa0, sa1, sb0, sb1):
        c = lax.axis_index("c")
        s = lax.axis_index("s")
        base = (c * 16 + s) * ETG
        pltpu.sync_copy(src_hbm.at[pl.ds(base, ETG)], src_v)
        pltpu.sync_copy(dst_hbm.at[pl.ds(base, ETG)], dst_v)

        abufs = (rows_a0, rows_a1)
        bbufs = (rows_b0, rows_b1)
        asems = (sa0, sa1)
        bsems = (sb0, sb1)
        for b in range(2):
            pltpu.async_copy(st_hbm.at[src_v.at[pl.ds(b * 128, 128)]],
                             abufs[b], asems[b])
            pltpu.async_copy(dt_hbm.at[dst_v.at[pl.ds(b * 128, 128)]],
                             bbufs[b], bsems[b])

        def outer(j2, _):
            for b in range(2):
                j = j2 * 2 + b
                pltpu.make_async_copy(
                    st_hbm.at[src_v.at[pl.ds(j * 128, 128)]], abufs[b],
                    asems[b]).wait()
                pltpu.sync_copy(abufs[b],
                                fa_hbm.at[pl.ds(base + j * 128, 128)])
                pltpu.make_async_copy(
                    dt_hbm.at[dst_v.at[pl.ds(j * 128, 128)]], bbufs[b],
                    bsems[b]).wait()
                pltpu.sync_copy(bbufs[b],
                                fb_hbm.at[pl.ds(base + j * 128, 128)])

                @pl.when(j + 2 < ETG // 128)
                def _():
                    pltpu.async_copy(
                        st_hbm.at[src_v.at[pl.ds((j + 2) * 128, 128)]],
                        abufs[b], asems[b])
                    pltpu.async_copy(
                        dt_hbm.at[dst_v.at[pl.ds((j + 2) * 128, 128)]],
                        bbufs[b], bsems[b])
            return 0
        lax.fori_loop(0, ETG // 128 // 2, outer, 0)

    run = pl.kernel(
        body,
        compiler_params=_SC_PARAMS,
        out_type=(jax.ShapeDtypeStruct((EP, 16), jnp.float32),
                  jax.ShapeDtypeStruct((EP, 16), jnp.float32)),
        mesh=_MESH,
        scratch_types=[
            pltpu.VMEM((ETG,), jnp.int32),
            pltpu.VMEM((ETG,), jnp.int32),
            pltpu.VMEM((128, 16), jnp.float32),
            pltpu.VMEM((128, 16), jnp.float32),
            pltpu.VMEM((128, 16), jnp.float32),
            pltpu.VMEM((128, 16), jnp.float32),
            pltpu.SemaphoreType.DMA,
            pltpu.SemaphoreType.DMA,
            pltpu.SemaphoreType.DMA,
            pltpu.SemaphoreType.DMA,
        ],
    )
    return run(st, dt, srcp, dstp)


def _mlp(fa, fb, rap, w1a, w1b, b1, p1a, p1b, p2, w2, b2):
    blk = 8192

    def body(fa_ref, fb_ref, ra_ref, w1a_ref, w1b_ref, b1_ref, p1a_ref,
             p1b_ref, p2_ref, w2_ref, b2_ref, pc_ref, md_ref):
        ha = _prelu(fa_ref[...], p1a_ref[...])
        hb = _prelu(fb_ref[...], p1b_ref[...])
        h = (jnp.dot(ha, w1a_ref[...], preferred_element_type=jnp.float32)
             + jnp.dot(hb, w1b_ref[...], preferred_element_type=jnp.float32)
             + b1_ref[...])
        h = _prelu(h, p2_ref[...])
        pc = jnp.dot(h, w2_ref[...], preferred_element_type=jnp.float32) + b2_ref[...]
        pc_ref[...] = pc
        md_ref[...] = ra_ref[...] * (2.0 * jax.nn.sigmoid(pc) - 1.0)

    return pl.pallas_call(
        body,
        grid=(EP // blk,),
        in_specs=[
            pl.BlockSpec((blk, 16), lambda i: (i, 0)),
            pl.BlockSpec((blk, 16), lambda i: (i, 0)),
            pl.BlockSpec((blk, 1), lambda i: (i, 0)),
            pl.BlockSpec((16, 8), lambda i: (0, 0)),
            pl.BlockSpec((16, 8), lambda i: (0, 0)),
            pl.BlockSpec((1, 8), lambda i: (0, 0)),
            pl.BlockSpec((1, 16), lambda i: (0, 0)),
            pl.BlockSpec((1, 16), lambda i: (0, 0)),
            pl.BlockSpec((1, 8), lambda i: (0, 0)),
            pl.BlockSpec((8, 1), lambda i: (0, 0)),
            pl.BlockSpec((1, 1), lambda i: (0, 0)),
        ],
        out_specs=[
            pl.BlockSpec((blk, 1), lambda i: (i, 0)),
            pl.BlockSpec((blk, 1), lambda i: (i, 0)),
        ],
        out_shape=[
            jax.ShapeDtypeStruct((EP, 1), jnp.float32),
            jax.ShapeDtypeStruct((EP, 1), jnp.float32),
        ],
    )(fa, fb, rap, w1a, w1b, b1, p1a, p1b, p2, w2, b2)


def _sc_segmean_src(md, srcp):
    def body(md_hbm, src_hbm, out_hbm,
             msg_v, src_v, den_v, deg_v, idx_v, den_sh, deg_sh):
        c = lax.axis_index("c")
        s = lax.axis_index("s")

        @pl.when(c == 0)
        def _():
            base = s * EPT
            zeros16 = jnp.zeros((16,), jnp.float32)
            iota16 = lax.iota(jnp.int32, 16)
            pltpu.sync_copy(md_hbm.at[pl.ds(base, EPT)], msg_v)
            pltpu.sync_copy(src_hbm.at[pl.ds(base, EPT)], src_v)

            def zero(i, _):
                den_v[pl.ds(i * 16, 16)] = zeros16
                deg_v[pl.ds(i * 16, 16)] = zeros16
                return 0
            lax.fori_loop(0, NP // 16, zero, 0)
            pltpu.sync_copy(den_v.at[pl.ds(0, NROWS_T)],
                            den_sh.at[pl.ds(s * NROWS_T, NROWS_T)])
            pltpu.sync_copy(deg_v.at[pl.ds(0, NROWS_T)],
                            deg_sh.at[pl.ds(s * NROWS_T, NROWS_T)])

            def p(i, _):
                si = src_v[pl.ds(i * 16, 16)]
                mdv = msg_v[pl.ds(i * 16, 16)]
                gid = base + i * 16 + iota16
                w = jnp.where(gid < E, 1.0, 0.0)
                plsc.addupdate_scatter(den_v, [si], mdv)
                plsc.addupdate_scatter(deg_v, [si], w)
                return 0
            lax.fori_loop(0, EPT // 16, p, 0)
            plsc.subcore_barrier()

            def dred(j, _):
                for k in range(8):
                    idx_v[0, pl.ds(k * 16, 16)] = j * 128 + k * 16 + iota16
                pltpu.sync_copy(den_v.at[pl.ds(j * 128, 128)],
                                den_sh.at[idx_v.at[0]], add=True)
                pltpu.sync_copy(deg_v.at[pl.ds(j * 128, 128)],
                                deg_sh.at[idx_v.at[0]], add=True)
                return 0
            lax.fori_loop(0, NP // 128, dred, 0)
            plsc.subcore_barrier()

            pltpu.sync_copy(den_sh.at[pl.ds(s * NROWS_T, NROWS_T)],
                            den_v.at[pl.ds(0, NROWS_T)])
            pltpu.sync_copy(deg_sh.at[pl.ds(s * NROWS_T, NROWS_T)],
                            deg_v.at[pl.ds(0, NROWS_T)])

            def fin(i, _):
                d = den_v[pl.ds(i * 16, 16)]
                g = deg_v[pl.ds(i * 16, 16)]
                den_v[pl.ds(i * 16, 16)] = d / jnp.maximum(g, 1.0)
                return 0
            lax.fori_loop(0, NROWS_T // 16, fin, 0)
            pltpu.sync_copy(den_v.at[pl.ds(0, NROWS_T)],
                            out_hbm.at[pl.ds(s * NROWS_T, NROWS_T)])

    run = pl.kernel(
        body,
        compiler_params=_SC_PARAMS,
        out_type=jax.ShapeDtypeStruct((NP,), jnp.float32),
        mesh=_MESH,
        scratch_types=[
            pltpu.VMEM((EPT,), jnp.float32),
            pltpu.VMEM((EPT,), jnp.int32),
            pltpu.VMEM((NP,), jnp.float32),
            pltpu.VMEM((NP,), jnp.float32),
            pltpu.VMEM((1, 128), jnp.int32),
            pltpu.VMEM_SHARED((NP,), jnp.float32),
            pltpu.VMEM_SHARED((NP,), jnp.float32),
        ],
    )
    return run(md, srcp)


def kernel(features, cluster_features, xws, yws, raw_affine, edge_index, params):
    src = edge_index[0]
    dst = edge_index[1]
    srcp = jnp.pad(src, (0, EP - E))
    dstp = jnp.pad(dst, (0, EP - E))

    parts = [jnp.pad(features, ((0, NP - N), (0, 0))),
             jnp.pad(cluster_features, ((0, NP - N), (0, 0)))]
    bparts = [jnp.zeros((1, 128), jnp.float32)] * 2
    relu_in = False

    for i in range(4):
        w = params['conv%d_W' % i]
        b = params['conv%d_b' % i]
        a_s = params['conv%d_asrc' % i][:, None]
        a_d = params['conv%d_adst' % i][:, None]
        npart = len(parts)
        dp = w.shape[0] // npart
        h4 = w.shape[1] // 4
        wparts = [w[k * dp:(k + 1) * dp] for k in range(npart)]
        *zq, el, er = _gat_linear(parts, wparts, bparts, a_s, a_d, relu_in)
        aggq = _sc_gat_edge(zq, el.reshape(NP), er.reshape(NP), srcp, dstp)
        parts = list(aggq)
        bparts = [b[k * h4:(k + 1) * h4][None, :] for k in range(4)]
        relu_in = True

    xy = jnp.zeros((NP, 16), jnp.float32)
    xy = xy.at[:N, 6].set(xws[:, 0]).at[:N, 7].set(yws[:, 0])
    sxyb = xy.at[:, 0:6].add(params['src_b'][None, :])
    dxyb = xy.at[:, 0:6].add(params['dst_b'][None, :])
    swp = jnp.zeros((128, 16), jnp.float32).at[:, 0:6].set(params['src_W'])
    dwp = jnp.zeros((128, 16), jnp.float32).at[:, 0:6].set(params['dst_W'])
    swparts = [swp[k * 32:(k + 1) * 32] for k in range(4)]
    dwparts = [dwp[k * 32:(k + 1) * 32] for k in range(4)]
    st, dt = _tables(parts, bparts, swparts, dwparts, sxyb, dxyb)

    fa, fb = _sc_gather_tables(st, dt, srcp, dstp)

    rap = jnp.pad(raw_affine, (0, EP - E))[:, None]
    w1 = params['cls1_W']
    w1a = jnp.zeros((16, 8), jnp.float32).at[0:8].set(w1[0:8])
    w1b = jnp.zeros((16, 8), jnp.float32).at[0:8].set(w1[8:16])
    p1 = params['prelu1']
    p1a = jnp.zeros((1, 16), jnp.float32).at[0, 0:8].set(p1[0:8])
    p1b = jnp.zeros((1, 16), jnp.float32).at[0, 0:8].set(p1[8:16])
    pcp, mdp = _mlp(fa, fb, rap, w1a, w1b, params['cls1_b'][None, :],
                    p1a, p1b, params['prelu2'][None, :],
                    params['cls2_W'], params['cls2_b'][None, :])

    pred_den = _sc_segmean_src(mdp.reshape(EP), srcp)
    return pcp[:E], pred_den[:N]


# speedup vs baseline: 14.7069x; 1.2295x over previous
import jax
import jax.numpy as jnp
from jax import lax
from jax.experimental import pallas as pl
from jax.experimental.pallas import tpu as pltpu
from jax.experimental.pallas import tpu_sc as plsc

N = 10000
NP = 10240
E = 160000
EP = 163840
EPT = EP // 16
ETG = EP // 32
NCHUNK = EPT // 128
NROWS_T = NP // 16

_MESH = plsc.VectorSubcoreMesh(
    core_axis_name="c", subcore_axis_name="s", num_cores=2, num_subcores=16)
_SC_PARAMS = pltpu.CompilerParams(needs_layout_passes=False,
                                  use_tc_tiling_on_sc=False)


def _prelu(v, a):
    return jnp.where(v >= 0, v, a * v)


def _gat_linear(parts, wparts, bparts, a_s, a_d, relu_in):
    nparts = len(parts)
    dps = [p.shape[1] for p in parts]
    dout = wparts[0].shape[1]
    h4 = dout // 8
    blk = 1024

    def body(*refs):
        part_refs = refs[:nparts]
        w_refs = refs[nparts:2 * nparts]
        b_refs = refs[2 * nparts:3 * nparts]
        as_ref = refs[3 * nparts]
        ad_ref = refs[3 * nparts + 1]
        zq_refs = refs[3 * nparts + 2:3 * nparts + 10]
        el_ref = refs[3 * nparts + 10]
        er_ref = refs[3 * nparts + 11]
        z = None
        for p_ref, w_ref, b_ref in zip(part_refs, w_refs, b_refs):
            a = p_ref[...]
            if relu_in:
                a = jnp.maximum(a + b_ref[...], 0.0)
            d = jnp.dot(a, w_ref[...], preferred_element_type=jnp.float32)
            z = d if z is None else z + d
        for q in range(8):
            zq_refs[q][...] = z[:, q * h4:(q + 1) * h4]
        el_ref[...] = jnp.dot(z, as_ref[...], preferred_element_type=jnp.float32)
        er_ref[...] = jnp.dot(z, ad_ref[...], preferred_element_type=jnp.float32)

    in_specs = (
        [pl.BlockSpec((blk, dp), lambda i: (i, 0)) for dp in dps]
        + [pl.BlockSpec((dp, dout), lambda i: (0, 0)) for dp in dps]
        + [pl.BlockSpec((1, dp), lambda i: (0, 0)) for dp in dps]
        + [pl.BlockSpec((dout, 1), lambda i: (0, 0))] * 2
    )
    out_specs = ([pl.BlockSpec((blk, h4), lambda i: (i, 0))] * 8
                 + [pl.BlockSpec((blk, 1), lambda i: (i, 0))] * 2)
    out_shape = ([jax.ShapeDtypeStruct((NP, h4), jnp.float32)] * 8
                 + [jax.ShapeDtypeStruct((NP, 1), jnp.float32)] * 2)
    return pl.pallas_call(
        body,
        grid=(NP // blk,),
        in_specs=in_specs,
        out_specs=out_specs,
        out_shape=out_shape,
    )(*parts, *wparts, *bparts, a_s, a_d)


def _sc_gat_edge(zq, el, er, srcp, dstp):
    h4 = zq[0].shape[1]

    def body(z0_hbm, z1_hbm, z2_hbm, z3_hbm, z4_hbm, z5_hbm, z6_hbm, z7_hbm,
             el_hbm, er_hbm, src_hbm, dst_hbm,
             a0_hbm, a1_hbm, a2_hbm, a3_hbm, a4_hbm, a5_hbm, a6_hbm, a7_hbm,
             el_v, er_v, src_v, dst_v, alpha_v,
             rows0_v, rows1_v, rows2_v, rows3_v,
             idx0_v, idx1_v, idx2_v, idx3_v, stage_v, vmax_v,
             gmax_sh, denom_sh, acc_sh, zst_sh,
             gs0, gs1, gs2, gs3, ss0, ss1, ss2, ss3):
        c = lax.axis_index("c")
        s = lax.axis_index("s")
        base = s * EPT
        zeros16 = jnp.zeros((16,), jnp.float32)
        iota16 = lax.iota(jnp.int32, 16)
        denom_v = el_v

        pltpu.sync_copy(el_hbm, el_v)
        pltpu.sync_copy(er_hbm, er_v)
        pltpu.sync_copy(src_hbm.at[pl.ds(base, EPT)], src_v)
        pltpu.sync_copy(dst_hbm.at[pl.ds(base, EPT)], dst_v)

        def p1(i, vmax):
            si = src_v[pl.ds(i * 16, 16)]
            di = dst_v[pl.ds(i * 16, 16)]
            e = plsc.load_gather(el_v, [si]) + plsc.load_gather(er_v, [di])
            e = jnp.where(e >= 0, e, 0.2 * e)
            alpha_v[pl.ds(i * 16, 16)] = e
            return jnp.maximum(vmax, e)
        vmax = lax.fori_loop(0, EPT // 16, p1,
                             jnp.full((16,), -3.0e38, jnp.float32))
        vmax_v[...] = vmax
        pltpu.sync_copy(vmax_v, gmax_sh.at[s])
        plsc.subcore_barrier()
        pltpu.sync_copy(gmax_sh, stage_v)
        m = stage_v[0]
        for t in range(1, 16):
            m = jnp.maximum(m, stage_v[t])
        gmax = jnp.max(m)

        def zden(i, _):
            denom_v[pl.ds(i * 16, 16)] = zeros16
            return 0
        lax.fori_loop(0, NP // 16, zden, 0)
        pltpu.sync_copy(denom_v.at[pl.ds(0, NROWS_T)],
                        denom_sh.at[pl.ds(s * NROWS_T, NROWS_T)])

        def p2(i, _):
            di = dst_v[pl.ds(i * 16, 16)]
            e = alpha_v[pl.ds(i * 16, 16)]
            gid = base + i * 16 + iota16
            ex = jnp.exp(e - gmax)
            ex = jnp.where(gid < E, ex, 0.0)
            alpha_v[pl.ds(i * 16, 16)] = ex
            plsc.addupdate_scatter(denom_v, [di], ex)
            return 0
        lax.fori_loop(0, EPT // 16, p2, 0)
        plsc.subcore_barrier()

        idxb = (idx0_v, idx1_v, idx2_v, idx3_v)
        dsem = (gs0, gs1, gs2, gs3)

        def dred(j4, _):
            for b in range(4):
                j = j4 * 4 + b
                for k in range(8):
                    idxb[b][0, pl.ds(k * 16, 16)] = j * 128 + k * 16 + iota16
                pltpu.async_copy(denom_v.at[pl.ds(j * 128, 128)],
                                 denom_sh.at[idxb[b].at[0]], dsem[b],
                                 add=True)
            for b in range(4):
                j = j4 * 4 + b
                pltpu.make_async_copy(denom_v.at[pl.ds(j * 128, 128)],
                                      denom_sh.at[idxb[b].at[0]],
                                      dsem[b]).wait()
            return 0
        lax.fori_loop(0, NP // 128 // 4, dred, 0)
        plsc.subcore_barrier()
        pltpu.sync_copy(denom_sh, denom_v)

        def p3(i, _):
            di = dst_v[pl.ds(i * 16, 16)]
            den = plsc.load_gather(denom_v, [di])
            alpha_v[pl.ds(i * 16, 16)] = (
                alpha_v[pl.ds(i * 16, 16)] / jnp.maximum(den, 1e-30))
            return 0
        lax.fori_loop(0, EPT // 16, p3, 0)

        bufs = (rows0_v, rows1_v, rows2_v, rows3_v)
        idxs = (idx0_v, idx1_v, idx2_v, idx3_v)
        gsems = (gs0, gs1, gs2, gs3)
        ssems = (ss0, ss1, ss2, ss3)

        def phase_b(z_hbm, agg_hbm):
            pltpu.sync_copy(z_hbm.at[pl.ds(s * NROWS_T, NROWS_T)],
                            zst_sh.at[pl.ds(s * NROWS_T, NROWS_T)])

            def zrow(i, _):
                for k in range(h4 // 16):
                    rows0_v[i, pl.ds(k * 16, 16)] = zeros16
                return 0
            lax.fori_loop(0, 128, zrow, 0)

            def zacc(j, _):
                pltpu.sync_copy(rows0_v,
                                acc_sh.at[pl.ds(s * NROWS_T + j * 128, 128)])
                return 0
            lax.fori_loop(0, NROWS_T // 128, zacc, 0)
            plsc.subcore_barrier()

            for b in range(2):
                pltpu.async_copy(zst_sh.at[src_v.at[pl.ds(b * 128, 128)]],
                                 bufs[b], gsems[b])

            def outer(j4, _):
                for b in range(4):
                    j = j4 * 4 + b
                    bg = (b + 2) % 4
                    pltpu.make_async_copy(
                        zst_sh.at[src_v.at[pl.ds(j * 128, 128)]], bufs[b],
                        gsems[b]).wait()

                    def grp(g, _):
                        av = alpha_v[pl.ds(j * 128 + g * 16, 16)]
                        for r in range(16):
                            arow = jnp.full((16,), av[r], jnp.float32)
                            row = g * 16 + r
                            for k in range(h4 // 16):
                                bufs[b][row, pl.ds(k * 16, 16)] = (
                                    bufs[b][row, pl.ds(k * 16, 16)] * arow)
                        return 0
                    lax.fori_loop(0, 8, grp, 0)
                    for k in range(8):
                        idxs[b][0, pl.ds(k * 16, 16)] = (
                            dst_v[pl.ds(j * 128 + k * 16, 16)])
                    pltpu.async_copy(bufs[b], acc_sh.at[idxs[b].at[0]],
                                     ssems[b], add=True)

                    @pl.when(j >= 2)
                    def _():
                        pltpu.make_async_copy(
                            bufs[bg], acc_sh.at[idxs[bg].at[0]],
                            ssems[bg]).wait()

                    @pl.when(j + 2 < NCHUNK)
                    def _():
                        pltpu.async_copy(
                            zst_sh.at[src_v.at[pl.ds((j + 2) * 128, 128)]],
                            bufs[bg], gsems[bg])
                return 0
            lax.fori_loop(0, NCHUNK // 4, outer, 0)
            for j in (NCHUNK - 2, NCHUNK - 1):
                b = j % 4
                pltpu.make_async_copy(bufs[b], acc_sh.at[idxs[b].at[0]],
                                      ssems[b]).wait()
            plsc.subcore_barrier()
            pltpu.sync_copy(acc_sh.at[pl.ds(s * NROWS_T, NROWS_T)],
                            agg_hbm.at[pl.ds(s * NROWS_T, NROWS_T)])

        @pl.when(c == 0)
        def _():
            phase_b(z0_hbm, a0_hbm)
            phase_b(z1_hbm, a1_hbm)
            phase_b(z2_hbm, a2_hbm)
            phase_b(z3_hbm, a3_hbm)

        @pl.when(c == 1)
        def _():
            phase_b(z4_hbm, a4_hbm)
            phase_b(z5_hbm, a5_hbm)
            phase_b(z6_hbm, a6_hbm)
            phase_b(z7_hbm, a7_hbm)

    run = pl.kernel(
        body,
        compiler_params=_SC_PARAMS,
        out_type=tuple(jax.ShapeDtypeStruct((NP, h4), jnp.float32)
                       for _ in range(8)),
        mesh=_MESH,
        scratch_types=[
            pltpu.VMEM((NP,), jnp.float32),
            pltpu.VMEM((NP,), jnp.float32),
            pltpu.VMEM((EPT,), jnp.int32),
            pltpu.VMEM((EPT,), jnp.int32),
            pltpu.VMEM((EPT,), jnp.float32),
            pltpu.VMEM((128, h4), jnp.float32),
            pltpu.VMEM((128, h4), jnp.float32),
            pltpu.VMEM((128, h4), jnp.float32),
            pltpu.VMEM((128, h4), jnp.float32),
            pltpu.VMEM((1, 128), jnp.int32),
            pltpu.VMEM((1, 128), jnp.int32),
            pltpu.VMEM((1, 128), jnp.int32),
            pltpu.VMEM((1, 128), jnp.int32),
            pltpu.VMEM((16, 16), jnp.float32),
            pltpu.VMEM((16,), jnp.float32),
            pltpu.VMEM_SHARED((16, 16), jnp.float32),
            pltpu.VMEM_SHARED((NP,), jnp.float32),
            pltpu.VMEM_SHARED((NP, h4), jnp.float32),
            pltpu.VMEM_SHARED((NP, h4), jnp.float32),
        ] + [pltpu.SemaphoreType.DMA] * 8,
    )
    return run(*zq, el, er, srcp, dstp)


def _tables(parts, bparts, swparts, dwparts, sxyb, dxyb):
    nparts = len(parts)
    dps = [p.shape[1] for p in parts]
    blk = 1024

    def body(*refs):
        part_refs = refs[:nparts]
        b_refs = refs[nparts:2 * nparts]
        sw_refs = refs[2 * nparts:3 * nparts]
        dw_refs = refs[3 * nparts:4 * nparts]
        sxy_ref = refs[4 * nparts]
        dxy_ref = refs[4 * nparts + 1]
        st_ref = refs[4 * nparts + 2]
        dt_ref = refs[4 * nparts + 3]
        st = sxy_ref[...]
        dt = dxy_ref[...]
        for p_ref, b_ref, sw_ref, dw_ref in zip(part_refs, b_refs, sw_refs,
                                                dw_refs):
            a = jnp.maximum(p_ref[...] + b_ref[...], 0.0)
            st = st + jnp.dot(a, sw_ref[...], preferred_element_type=jnp.float32)
            dt = dt + jnp.dot(a, dw_ref[...], preferred_element_type=jnp.float32)
        st_ref[...] = st
        dt_ref[...] = dt

    in_specs = (
        [pl.BlockSpec((blk, dp), lambda i: (i, 0)) for dp in dps]
        + [pl.BlockSpec((1, dp), lambda i: (0, 0)) for dp in dps]
        + [pl.BlockSpec((dp, 16), lambda i: (0, 0)) for dp in dps] * 2
        + [pl.BlockSpec((blk, 16), lambda i: (i, 0))] * 2
    )
    return pl.pallas_call(
        body,
        grid=(NP // blk,),
        in_specs=in_specs,
        out_specs=[pl.BlockSpec((blk, 16), lambda i: (i, 0))] * 2,
        out_shape=[jax.ShapeDtypeStruct((NP, 16), jnp.float32)] * 2,
    )(*parts, *bparts, *swparts, *dwparts, sxyb, dxyb)


def _sc_gather_tables(st, dt, srcp, dstp):
    def body(st_hbm, dt_hbm, src_hbm, dst_hbm, fa_hbm, fb_hbm,
             src_v, dst_v, rows_a0, rows_a1, rows_b0, rows_b1,
             sa0, sa1, sb0, sb1):
        c = lax.axis_index("c")
        s = lax.axis_index("s")
        base = (c * 16 + s) * ETG
        pltpu.sync_copy(src_hbm.at[pl.ds(base, ETG)], src_v)
        pltpu.sync_copy(dst_hbm.at[pl.ds(base, ETG)], dst_v)

        abufs = (rows_a0, rows_a1)
        bbufs = (rows_b0, rows_b1)
        asems = (sa0, sa1)
        bsems = (sb0, sb1)
        for b in range(2):
            pltpu.async_copy(st_hbm.at[src_v.at[pl.ds(b * 128, 128)]],
                             abufs[b], asems[b])
            pltpu.async_copy(dt_hbm.at[dst_v.at[pl.ds(b * 128, 128)]],
                             bbufs[b], bsems[b])

        def outer(j2, _):
            for b in range(2):
                j = j2 * 2 + b
                pltpu.make_async_copy(
                    st_hbm.at[src_v.at[pl.ds(j * 128, 128)]], abufs[b],
                    asems[b]).wait()
                pltpu.sync_copy(abufs[b],
                                fa_hbm.at[pl.ds(base + j * 128, 128)])
                pltpu.make_async_copy(
                    dt_hbm.at[dst_v.at[pl.ds(j * 128, 128)]], bbufs[b],
                    bsems[b]).wait()
                pltpu.sync_copy(bbufs[b],
                                fb_hbm.at[pl.ds(base + j * 128, 128)])

                @pl.when(j + 2 < ETG // 128)
                def _():
                    pltpu.async_copy(
                        st_hbm.at[src_v.at[pl.ds((j + 2) * 128, 128)]],
                        abufs[b], asems[b])
                    pltpu.async_copy(
                        dt_hbm.at[dst_v.at[pl.ds((j + 2) * 128, 128)]],
                        bbufs[b], bsems[b])
            return 0
        lax.fori_loop(0, ETG // 128 // 2, outer, 0)

    run = pl.kernel(
        body,
        compiler_params=_SC_PARAMS,
        out_type=(jax.ShapeDtypeStruct((EP, 16), jnp.float32),
                  jax.ShapeDtypeStruct((EP, 16), jnp.float32)),
        mesh=_MESH,
        scratch_types=[
            pltpu.VMEM((ETG,), jnp.int32),
            pltpu.VMEM((ETG,), jnp.int32),
            pltpu.VMEM((128, 16), jnp.float32),
            pltpu.VMEM((128, 16), jnp.float32),
            pltpu.VMEM((128, 16), jnp.float32),
            pltpu.VMEM((128, 16), jnp.float32),
            pltpu.SemaphoreType.DMA,
            pltpu.SemaphoreType.DMA,
            pltpu.SemaphoreType.DMA,
            pltpu.SemaphoreType.DMA,
        ],
    )
    return run(st, dt, srcp, dstp)


def _mlp(fa, fb, rap, w1a, w1b, b1, p1a, p1b, p2, w2, b2):
    blk = 8192

    def body(fa_ref, fb_ref, ra_ref, w1a_ref, w1b_ref, b1_ref, p1a_ref,
             p1b_ref, p2_ref, w2_ref, b2_ref, pc_ref, md_ref):
        ha = _prelu(fa_ref[...], p1a_ref[...])
        hb = _prelu(fb_ref[...], p1b_ref[...])
        h = (jnp.dot(ha, w1a_ref[...], preferred_element_type=jnp.float32)
             + jnp.dot(hb, w1b_ref[...], preferred_element_type=jnp.float32)
             + b1_ref[...])
        h = _prelu(h, p2_ref[...])
        pc = jnp.dot(h, w2_ref[...], preferred_element_type=jnp.float32) + b2_ref[...]
        pc_ref[...] = pc
        md_ref[...] = ra_ref[...] * (2.0 * jax.nn.sigmoid(pc) - 1.0)

    return pl.pallas_call(
        body,
        grid=(EP // blk,),
        in_specs=[
            pl.BlockSpec((blk, 16), lambda i: (i, 0)),
            pl.BlockSpec((blk, 16), lambda i: (i, 0)),
            pl.BlockSpec((blk, 1), lambda i: (i, 0)),
            pl.BlockSpec((16, 8), lambda i: (0, 0)),
            pl.BlockSpec((16, 8), lambda i: (0, 0)),
            pl.BlockSpec((1, 8), lambda i: (0, 0)),
            pl.BlockSpec((1, 16), lambda i: (0, 0)),
            pl.BlockSpec((1, 16), lambda i: (0, 0)),
            pl.BlockSpec((1, 8), lambda i: (0, 0)),
            pl.BlockSpec((8, 1), lambda i: (0, 0)),
            pl.BlockSpec((1, 1), lambda i: (0, 0)),
        ],
        out_specs=[
            pl.BlockSpec((blk, 1), lambda i: (i, 0)),
            pl.BlockSpec((blk, 1), lambda i: (i, 0)),
        ],
        out_shape=[
            jax.ShapeDtypeStruct((EP, 1), jnp.float32),
            jax.ShapeDtypeStruct((EP, 1), jnp.float32),
        ],
    )(fa, fb, rap, w1a, w1b, b1, p1a, p1b, p2, w2, b2)


def _sc_segmean_src(md, srcp):
    def body(md_hbm, src_hbm, out_hbm,
             msg_v, src_v, den_v, deg_v, idx_v, den_sh, deg_sh):
        c = lax.axis_index("c")
        s = lax.axis_index("s")

        @pl.when(c == 0)
        def _():
            base = s * EPT
            zeros16 = jnp.zeros((16,), jnp.float32)
            iota16 = lax.iota(jnp.int32, 16)
            pltpu.sync_copy(md_hbm.at[pl.ds(base, EPT)], msg_v)
            pltpu.sync_copy(src_hbm.at[pl.ds(base, EPT)], src_v)

            def zero(i, _):
                den_v[pl.ds(i * 16, 16)] = zeros16
                deg_v[pl.ds(i * 16, 16)] = zeros16
                return 0
            lax.fori_loop(0, NP // 16, zero, 0)
            pltpu.sync_copy(den_v.at[pl.ds(0, NROWS_T)],
                            den_sh.at[pl.ds(s * NROWS_T, NROWS_T)])
            pltpu.sync_copy(deg_v.at[pl.ds(0, NROWS_T)],
                            deg_sh.at[pl.ds(s * NROWS_T, NROWS_T)])

            def p(i, _):
                si = src_v[pl.ds(i * 16, 16)]
                mdv = msg_v[pl.ds(i * 16, 16)]
                gid = base + i * 16 + iota16
                w = jnp.where(gid < E, 1.0, 0.0)
                plsc.addupdate_scatter(den_v, [si], mdv)
                plsc.addupdate_scatter(deg_v, [si], w)
                return 0
            lax.fori_loop(0, EPT // 16, p, 0)
            plsc.subcore_barrier()

            def dred(j, _):
                for k in range(8):
                    idx_v[0, pl.ds(k * 16, 16)] = j * 128 + k * 16 + iota16
                pltpu.sync_copy(den_v.at[pl.ds(j * 128, 128)],
                                den_sh.at[idx_v.at[0]], add=True)
                pltpu.sync_copy(deg_v.at[pl.ds(j * 128, 128)],
                                deg_sh.at[idx_v.at[0]], add=True)
                return 0
            lax.fori_loop(0, NP // 128, dred, 0)
            plsc.subcore_barrier()

            pltpu.sync_copy(den_sh.at[pl.ds(s * NROWS_T, NROWS_T)],
                            den_v.at[pl.ds(0, NROWS_T)])
            pltpu.sync_copy(deg_sh.at[pl.ds(s * NROWS_T, NROWS_T)],
                            deg_v.at[pl.ds(0, NROWS_T)])

            def fin(i, _):
                d = den_v[pl.ds(i * 16, 16)]
                g = deg_v[pl.ds(i * 16, 16)]
                den_v[pl.ds(i * 16, 16)] = d / jnp.maximum(g, 1.0)
                return 0
            lax.fori_loop(0, NROWS_T // 16, fin, 0)
            pltpu.sync_copy(den_v.at[pl.ds(0, NROWS_T)],
                            out_hbm.at[pl.ds(s * NROWS_T, NROWS_T)])

    run = pl.kernel(
        body,
        compiler_params=_SC_PARAMS,
        out_type=jax.ShapeDtypeStruct((NP,), jnp.float32),
        mesh=_MESH,
        scratch_types=[
            pltpu.VMEM((EPT,), jnp.float32),
            pltpu.VMEM((EPT,), jnp.int32),
            pltpu.VMEM((NP,), jnp.float32),
            pltpu.VMEM((NP,), jnp.float32),
            pltpu.VMEM((1, 128), jnp.int32),
            pltpu.VMEM_SHARED((NP,), jnp.float32),
            pltpu.VMEM_SHARED((NP,), jnp.float32),
        ],
    )
    return run(md, srcp)


def kernel(features, cluster_features, xws, yws, raw_affine, edge_index, params):
    src = edge_index[0]
    dst = edge_index[1]
    srcp = jnp.pad(src, (0, EP - E))
    dstp = jnp.pad(dst, (0, EP - E))

    parts = [jnp.pad(features, ((0, NP - N), (0, 0))),
             jnp.pad(cluster_features, ((0, NP - N), (0, 0)))]
    bparts = [jnp.zeros((1, 128), jnp.float32)] * 2
    relu_in = False

    for i in range(4):
        w = params['conv%d_W' % i]
        b = params['conv%d_b' % i]
        a_s = params['conv%d_asrc' % i][:, None]
        a_d = params['conv%d_adst' % i][:, None]
        npart = len(parts)
        dp = w.shape[0] // npart
        h4 = w.shape[1] // 8
        wparts = [w[k * dp:(k + 1) * dp] for k in range(npart)]
        *zq, el, er = _gat_linear(parts, wparts, bparts, a_s, a_d, relu_in)
        aggq = _sc_gat_edge(zq, el.reshape(NP), er.reshape(NP), srcp, dstp)
        parts = list(aggq)
        bparts = [b[k * h4:(k + 1) * h4][None, :] for k in range(8)]
        relu_in = True

    xy = jnp.zeros((NP, 16), jnp.float32)
    xy = xy.at[:N, 6].set(xws[:, 0]).at[:N, 7].set(yws[:, 0])
    sxyb = xy.at[:, 0:6].add(params['src_b'][None, :])
    dxyb = xy.at[:, 0:6].add(params['dst_b'][None, :])
    swp = jnp.zeros((128, 16), jnp.float32).at[:, 0:6].set(params['src_W'])
    dwp = jnp.zeros((128, 16), jnp.float32).at[:, 0:6].set(params['dst_W'])
    swparts = [swp[k * 16:(k + 1) * 16] for k in range(8)]
    dwparts = [dwp[k * 16:(k + 1) * 16] for k in range(8)]
    st, dt = _tables(parts, bparts, swparts, dwparts, sxyb, dxyb)

    fa, fb = _sc_gather_tables(st, dt, srcp, dstp)

    rap = jnp.pad(raw_affine, (0, EP - E))[:, None]
    w1 = params['cls1_W']
    w1a = jnp.zeros((16, 8), jnp.float32).at[0:8].set(w1[0:8])
    w1b = jnp.zeros((16, 8), jnp.float32).at[0:8].set(w1[8:16])
    p1 = params['prelu1']
    p1a = jnp.zeros((1, 16), jnp.float32).at[0, 0:8].set(p1[0:8])
    p1b = jnp.zeros((1, 16), jnp.float32).at[0, 0:8].set(p1[8:16])
    pcp, mdp = _mlp(fa, fb, rap, w1a, w1b, params['cls1_b'][None, :],
                    p1a, p1b, params['prelu2'][None, :],
                    params['cls2_W'], params['cls2_b'][None, :])

    pred_den = _sc_segmean_src(mdp.reshape(EP), srcp)
    return pcp[:E], pred_den[:N]


# speedup vs baseline: 14.9012x; 1.0132x over previous
import jax
import jax.numpy as jnp
from jax import lax
from jax.experimental import pallas as pl
from jax.experimental.pallas import tpu as pltpu
from jax.experimental.pallas import tpu_sc as plsc

N = 10000
NP = 10240
E = 160000
EP = 163840
EPT = EP // 16
ETG = EP // 32
NCHUNK = EPT // 128
NROWS_T = NP // 16

_MESH = plsc.VectorSubcoreMesh(
    core_axis_name="c", subcore_axis_name="s", num_cores=2, num_subcores=16)
_SC_PARAMS = pltpu.CompilerParams(needs_layout_passes=False,
                                  use_tc_tiling_on_sc=False)


def _prelu(v, a):
    return jnp.where(v >= 0, v, a * v)


def _gat_linear(parts, wparts, bparts, a_s, a_d, relu_in):
    nparts = len(parts)
    dps = [p.shape[1] for p in parts]
    dout = wparts[0].shape[1]
    h4 = dout // 8
    blk = 1024

    def body(*refs):
        part_refs = refs[:nparts]
        w_refs = refs[nparts:2 * nparts]
        b_refs = refs[2 * nparts:3 * nparts]
        as_ref = refs[3 * nparts]
        ad_ref = refs[3 * nparts + 1]
        zq_refs = refs[3 * nparts + 2:3 * nparts + 10]
        el_ref = refs[3 * nparts + 10]
        er_ref = refs[3 * nparts + 11]
        z = None
        for p_ref, w_ref, b_ref in zip(part_refs, w_refs, b_refs):
            a = p_ref[...]
            if relu_in:
                a = jnp.maximum(a + b_ref[...], 0.0)
            d = jnp.dot(a, w_ref[...], preferred_element_type=jnp.float32)
            z = d if z is None else z + d
        for q in range(8):
            zq_refs[q][...] = z[:, q * h4:(q + 1) * h4]
        el_ref[...] = jnp.dot(z, as_ref[...], preferred_element_type=jnp.float32)
        er_ref[...] = jnp.dot(z, ad_ref[...], preferred_element_type=jnp.float32)

    in_specs = (
        [pl.BlockSpec((blk, dp), lambda i: (i, 0)) for dp in dps]
        + [pl.BlockSpec((dp, dout), lambda i: (0, 0)) for dp in dps]
        + [pl.BlockSpec((1, dp), lambda i: (0, 0)) for dp in dps]
        + [pl.BlockSpec((dout, 1), lambda i: (0, 0))] * 2
    )
    out_specs = ([pl.BlockSpec((blk, h4), lambda i: (i, 0))] * 8
                 + [pl.BlockSpec((blk, 1), lambda i: (i, 0))] * 2)
    out_shape = ([jax.ShapeDtypeStruct((NP, h4), jnp.float32)] * 8
                 + [jax.ShapeDtypeStruct((NP, 1), jnp.float32)] * 2)
    return pl.pallas_call(
        body,
        grid=(NP // blk,),
        in_specs=in_specs,
        out_specs=out_specs,
        out_shape=out_shape,
    )(*parts, *wparts, *bparts, a_s, a_d)


def _sc_gat_edge(zq, el, er, srcp, dstp):
    h4 = zq[0].shape[1]

    def body(z0_hbm, z1_hbm, z2_hbm, z3_hbm, z4_hbm, z5_hbm, z6_hbm, z7_hbm,
             el_hbm, er_hbm, src_hbm, dst_hbm,
             a0_hbm, a1_hbm, a2_hbm, a3_hbm, a4_hbm, a5_hbm, a6_hbm, a7_hbm,
             el_v, er_v, src_v, dst_v, alpha_v,
             rows0_v, rows1_v, rows2_v, rows3_v,
             idx0_v, idx1_v, idx2_v, idx3_v, stage_v, vmax_v,
             gmax_sh, denom_sh, acc_sh, zst_sh,
             gs0, gs1, gs2, gs3, ss0, ss1, ss2, ss3):
        c = lax.axis_index("c")
        s = lax.axis_index("s")
        base = s * EPT
        zeros16 = jnp.zeros((16,), jnp.float32)
        iota16 = lax.iota(jnp.int32, 16)
        denom_v = el_v

        pltpu.sync_copy(el_hbm, el_v)
        pltpu.sync_copy(er_hbm, er_v)
        pltpu.sync_copy(src_hbm.at[pl.ds(base, EPT)], src_v)
        pltpu.sync_copy(dst_hbm.at[pl.ds(base, EPT)], dst_v)

        def p1(i, vmax):
            si = src_v[pl.ds(i * 16, 16)]
            di = dst_v[pl.ds(i * 16, 16)]
            e = plsc.load_gather(el_v, [si]) + plsc.load_gather(er_v, [di])
            e = jnp.where(e >= 0, e, 0.2 * e)
            alpha_v[pl.ds(i * 16, 16)] = e
            return jnp.maximum(vmax, e)
        vmax = lax.fori_loop(0, EPT // 16, p1,
                             jnp.full((16,), -3.0e38, jnp.float32))
        vmax_v[...] = vmax
        pltpu.sync_copy(vmax_v, gmax_sh.at[s])
        plsc.subcore_barrier()
        pltpu.sync_copy(gmax_sh, stage_v)
        m = stage_v[0]
        for t in range(1, 16):
            m = jnp.maximum(m, stage_v[t])
        gmax = jnp.max(m)

        def zden(i, _):
            denom_v[pl.ds(i * 16, 16)] = zeros16
            return 0
        lax.fori_loop(0, NP // 16, zden, 0)
        pltpu.sync_copy(denom_v.at[pl.ds(0, NROWS_T)],
                        denom_sh.at[pl.ds(s * NROWS_T, NROWS_T)])

        def p2(i, _):
            di = dst_v[pl.ds(i * 16, 16)]
            e = alpha_v[pl.ds(i * 16, 16)]
            gid = base + i * 16 + iota16
            ex = jnp.exp(e - gmax)
            ex = jnp.where(gid < E, ex, 0.0)
            alpha_v[pl.ds(i * 16, 16)] = ex
            plsc.addupdate_scatter(denom_v, [di], ex)
            return 0
        lax.fori_loop(0, EPT // 16, p2, 0)
        plsc.subcore_barrier()

        idxb = (idx0_v, idx1_v, idx2_v, idx3_v)
        dsem = (gs0, gs1, gs2, gs3)

        def dred(j4, _):
            for b in range(4):
                j = j4 * 4 + b
                for k in range(8):
                    idxb[b][0, pl.ds(k * 16, 16)] = j * 128 + k * 16 + iota16
                pltpu.async_copy(denom_v.at[pl.ds(j * 128, 128)],
                                 denom_sh.at[idxb[b].at[0]], dsem[b],
                                 add=True)
            for b in range(4):
                j = j4 * 4 + b
                pltpu.make_async_copy(denom_v.at[pl.ds(j * 128, 128)],
                                      denom_sh.at[idxb[b].at[0]],
                                      dsem[b]).wait()
            return 0
        lax.fori_loop(0, NP // 128 // 4, dred, 0)
        plsc.subcore_barrier()
        pltpu.sync_copy(denom_sh, denom_v)

        def p3(i, _):
            di = dst_v[pl.ds(i * 16, 16)]
            den = plsc.load_gather(denom_v, [di])
            alpha_v[pl.ds(i * 16, 16)] = (
                alpha_v[pl.ds(i * 16, 16)] / jnp.maximum(den, 1e-30))
            return 0
        lax.fori_loop(0, EPT // 16, p3, 0)

        bufs = (rows0_v, rows1_v, rows2_v, rows3_v)
        idxs = (idx0_v, idx1_v, idx2_v, idx3_v)
        gsems = (gs0, gs1, gs2, gs3)
        ssems = (ss0, ss1, ss2, ss3)

        def phase_b(z_hbm, agg_hbm):
            pltpu.sync_copy(z_hbm.at[pl.ds(s * NROWS_T, NROWS_T)],
                            zst_sh.at[pl.ds(s * NROWS_T, NROWS_T)])

            def zrow(i, _):
                for k in range(h4 // 16):
                    rows0_v[i, pl.ds(k * 16, 16)] = zeros16
                return 0
            lax.fori_loop(0, 128, zrow, 0)

            def zacc(j, _):
                pltpu.sync_copy(rows0_v,
                                acc_sh.at[pl.ds(s * NROWS_T + j * 128, 128)])
                return 0
            lax.fori_loop(0, NROWS_T // 128, zacc, 0)
            plsc.subcore_barrier()

            for b in range(2):
                pltpu.async_copy(zst_sh.at[src_v.at[pl.ds(b * 128, 128)]],
                                 bufs[b], gsems[b])

            def outer(j4, _):
                for b in range(4):
                    j = j4 * 4 + b
                    bg = (b + 2) % 4
                    pltpu.make_async_copy(
                        zst_sh.at[src_v.at[pl.ds(j * 128, 128)]], bufs[b],
                        gsems[b]).wait()

                    def grp(g, _):
                        av = alpha_v[pl.ds(j * 128 + g * 16, 16)]
                        for r in range(16):
                            arow = jnp.full((16,), av[r], jnp.float32)
                            row = g * 16 + r
                            for k in range(h4 // 16):
                                bufs[b][row, pl.ds(k * 16, 16)] = (
                                    bufs[b][row, pl.ds(k * 16, 16)] * arow)
                        return 0
                    lax.fori_loop(0, 8, grp, 0)
                    for k in range(8):
                        idxs[b][0, pl.ds(k * 16, 16)] = (
                            dst_v[pl.ds(j * 128 + k * 16, 16)])
                    pltpu.async_copy(bufs[b], acc_sh.at[idxs[b].at[0]],
                                     ssems[b], add=True)

                    @pl.when(j >= 2)
                    def _():
                        pltpu.make_async_copy(
                            bufs[bg], acc_sh.at[idxs[bg].at[0]],
                            ssems[bg]).wait()

                    @pl.when(j + 2 < NCHUNK)
                    def _():
                        pltpu.async_copy(
                            zst_sh.at[src_v.at[pl.ds((j + 2) * 128, 128)]],
                            bufs[bg], gsems[bg])
                return 0
            lax.fori_loop(0, NCHUNK // 4, outer, 0)
            for j in (NCHUNK - 2, NCHUNK - 1):
                b = j % 4
                pltpu.make_async_copy(bufs[b], acc_sh.at[idxs[b].at[0]],
                                      ssems[b]).wait()
            plsc.subcore_barrier()
            pltpu.sync_copy(acc_sh.at[pl.ds(s * NROWS_T, NROWS_T)],
                            agg_hbm.at[pl.ds(s * NROWS_T, NROWS_T)])

        @pl.when(c == 0)
        def _():
            phase_b(z0_hbm, a0_hbm)
            phase_b(z1_hbm, a1_hbm)
            phase_b(z2_hbm, a2_hbm)
            phase_b(z3_hbm, a3_hbm)

        @pl.when(c == 1)
        def _():
            phase_b(z4_hbm, a4_hbm)
            phase_b(z5_hbm, a5_hbm)
            phase_b(z6_hbm, a6_hbm)
            phase_b(z7_hbm, a7_hbm)

    run = pl.kernel(
        body,
        compiler_params=_SC_PARAMS,
        out_type=tuple(jax.ShapeDtypeStruct((NP, h4), jnp.float32)
                       for _ in range(8)),
        mesh=_MESH,
        scratch_types=[
            pltpu.VMEM((NP,), jnp.float32),
            pltpu.VMEM((NP,), jnp.float32),
            pltpu.VMEM((EPT,), jnp.int32),
            pltpu.VMEM((EPT,), jnp.int32),
            pltpu.VMEM((EPT,), jnp.float32),
            pltpu.VMEM((128, h4), jnp.float32),
            pltpu.VMEM((128, h4), jnp.float32),
            pltpu.VMEM((128, h4), jnp.float32),
            pltpu.VMEM((128, h4), jnp.float32),
            pltpu.VMEM((1, 128), jnp.int32),
            pltpu.VMEM((1, 128), jnp.int32),
            pltpu.VMEM((1, 128), jnp.int32),
            pltpu.VMEM((1, 128), jnp.int32),
            pltpu.VMEM((16, 16), jnp.float32),
            pltpu.VMEM((16,), jnp.float32),
            pltpu.VMEM_SHARED((16, 16), jnp.float32),
            pltpu.VMEM_SHARED((NP,), jnp.float32),
            pltpu.VMEM_SHARED((NP, h4), jnp.float32),
            pltpu.VMEM_SHARED((NP, h4), jnp.float32),
        ] + [pltpu.SemaphoreType.DMA] * 8,
    )
    return run(*zq, el, er, srcp, dstp)


def _tables(parts, bparts, swparts, dwparts, sxyb, dxyb):
    nparts = len(parts)
    dps = [p.shape[1] for p in parts]
    blk = 1024

    def body(*refs):
        part_refs = refs[:nparts]
        b_refs = refs[nparts:2 * nparts]
        sw_refs = refs[2 * nparts:3 * nparts]
        dw_refs = refs[3 * nparts:4 * nparts]
        sxy_ref = refs[4 * nparts]
        dxy_ref = refs[4 * nparts + 1]
        st_ref = refs[4 * nparts + 2]
        dt_ref = refs[4 * nparts + 3]
        st = sxy_ref[...]
        dt = dxy_ref[...]
        for p_ref, b_ref, sw_ref, dw_ref in zip(part_refs, b_refs, sw_refs,
                                                dw_refs):
            a = jnp.maximum(p_ref[...] + b_ref[...], 0.0)
            st = st + jnp.dot(a, sw_ref[...], preferred_element_type=jnp.float32)
            dt = dt + jnp.dot(a, dw_ref[...], preferred_element_type=jnp.float32)
        st_ref[...] = st
        dt_ref[...] = dt

    in_specs = (
        [pl.BlockSpec((blk, dp), lambda i: (i, 0)) for dp in dps]
        + [pl.BlockSpec((1, dp), lambda i: (0, 0)) for dp in dps]
        + [pl.BlockSpec((dp, 16), lambda i: (0, 0)) for dp in dps] * 2
        + [pl.BlockSpec((blk, 16), lambda i: (i, 0))] * 2
    )
    return pl.pallas_call(
        body,
        grid=(NP // blk,),
        in_specs=in_specs,
        out_specs=[pl.BlockSpec((blk, 16), lambda i: (i, 0))] * 2,
        out_shape=[jax.ShapeDtypeStruct((NP, 16), jnp.float32)] * 2,
    )(*parts, *bparts, *swparts, *dwparts, sxyb, dxyb)


def _sc_gather_tables(st, dt, srcp, dstp):
    def body(st_hbm, dt_hbm, src_hbm, dst_hbm, fa_hbm, fb_hbm,
             src_v, dst_v, rows_a0, rows_a1, rows_b0, rows_b1,
             st_sh, dt_sh, sa0, sa1, sb0, sb1):
        c = lax.axis_index("c")
        s = lax.axis_index("s")
        base = (c * 16 + s) * ETG
        pltpu.sync_copy(st_hbm.at[pl.ds(s * NROWS_T, NROWS_T)],
                        st_sh.at[pl.ds(s * NROWS_T, NROWS_T)])
        pltpu.sync_copy(dt_hbm.at[pl.ds(s * NROWS_T, NROWS_T)],
                        dt_sh.at[pl.ds(s * NROWS_T, NROWS_T)])
        pltpu.sync_copy(src_hbm.at[pl.ds(base, ETG)], src_v)
        pltpu.sync_copy(dst_hbm.at[pl.ds(base, ETG)], dst_v)
        plsc.subcore_barrier()

        abufs = (rows_a0, rows_a1)
        bbufs = (rows_b0, rows_b1)
        asems = (sa0, sa1)
        bsems = (sb0, sb1)
        for b in range(2):
            pltpu.async_copy(st_sh.at[src_v.at[pl.ds(b * 128, 128)]],
                             abufs[b], asems[b])
            pltpu.async_copy(dt_sh.at[dst_v.at[pl.ds(b * 128, 128)]],
                             bbufs[b], bsems[b])

        def outer(j2, _):
            for b in range(2):
                j = j2 * 2 + b
                pltpu.make_async_copy(
                    st_sh.at[src_v.at[pl.ds(j * 128, 128)]], abufs[b],
                    asems[b]).wait()
                pltpu.sync_copy(abufs[b],
                                fa_hbm.at[pl.ds(base + j * 128, 128)])
                pltpu.make_async_copy(
                    dt_sh.at[dst_v.at[pl.ds(j * 128, 128)]], bbufs[b],
                    bsems[b]).wait()
                pltpu.sync_copy(bbufs[b],
                                fb_hbm.at[pl.ds(base + j * 128, 128)])

                @pl.when(j + 2 < ETG // 128)
                def _():
                    pltpu.async_copy(
                        st_sh.at[src_v.at[pl.ds((j + 2) * 128, 128)]],
                        abufs[b], asems[b])
                    pltpu.async_copy(
                        dt_sh.at[dst_v.at[pl.ds((j + 2) * 128, 128)]],
                        bbufs[b], bsems[b])
            return 0
        lax.fori_loop(0, ETG // 128 // 2, outer, 0)

    run = pl.kernel(
        body,
        compiler_params=_SC_PARAMS,
        out_type=(jax.ShapeDtypeStruct((EP, 16), jnp.float32),
                  jax.ShapeDtypeStruct((EP, 16), jnp.float32)),
        mesh=_MESH,
        scratch_types=[
            pltpu.VMEM((ETG,), jnp.int32),
            pltpu.VMEM((ETG,), jnp.int32),
            pltpu.VMEM((128, 16), jnp.float32),
            pltpu.VMEM((128, 16), jnp.float32),
            pltpu.VMEM((128, 16), jnp.float32),
            pltpu.VMEM((128, 16), jnp.float32),
            pltpu.VMEM_SHARED((NP, 16), jnp.float32),
            pltpu.VMEM_SHARED((NP, 16), jnp.float32),
            pltpu.SemaphoreType.DMA,
            pltpu.SemaphoreType.DMA,
            pltpu.SemaphoreType.DMA,
            pltpu.SemaphoreType.DMA,
        ],
    )
    return run(st, dt, srcp, dstp)


def _mlp(fa, fb, rap, w1a, w1b, b1, p1a, p1b, p2, w2, b2):
    blk = 8192

    def body(fa_ref, fb_ref, ra_ref, w1a_ref, w1b_ref, b1_ref, p1a_ref,
             p1b_ref, p2_ref, w2_ref, b2_ref, pc_ref, md_ref):
        ha = _prelu(fa_ref[...], p1a_ref[...])
        hb = _prelu(fb_ref[...], p1b_ref[...])
        h = (jnp.dot(ha, w1a_ref[...], preferred_element_type=jnp.float32)
             + jnp.dot(hb, w1b_ref[...], preferred_element_type=jnp.float32)
             + b1_ref[...])
        h = _prelu(h, p2_ref[...])
        pc = jnp.dot(h, w2_ref[...], preferred_element_type=jnp.float32) + b2_ref[...]
        pc_ref[...] = pc
        md_ref[...] = ra_ref[...] * (2.0 * jax.nn.sigmoid(pc) - 1.0)

    return pl.pallas_call(
        body,
        grid=(EP // blk,),
        in_specs=[
            pl.BlockSpec((blk, 16), lambda i: (i, 0)),
            pl.BlockSpec((blk, 16), lambda i: (i, 0)),
            pl.BlockSpec((blk, 1), lambda i: (i, 0)),
            pl.BlockSpec((16, 8), lambda i: (0, 0)),
            pl.BlockSpec((16, 8), lambda i: (0, 0)),
            pl.BlockSpec((1, 8), lambda i: (0, 0)),
            pl.BlockSpec((1, 16), lambda i: (0, 0)),
            pl.BlockSpec((1, 16), lambda i: (0, 0)),
            pl.BlockSpec((1, 8), lambda i: (0, 0)),
            pl.BlockSpec((8, 1), lambda i: (0, 0)),
            pl.BlockSpec((1, 1), lambda i: (0, 0)),
        ],
        out_specs=[
            pl.BlockSpec((blk, 1), lambda i: (i, 0)),
            pl.BlockSpec((blk, 1), lambda i: (i, 0)),
        ],
        out_shape=[
            jax.ShapeDtypeStruct((EP, 1), jnp.float32),
            jax.ShapeDtypeStruct((EP, 1), jnp.float32),
        ],
    )(fa, fb, rap, w1a, w1b, b1, p1a, p1b, p2, w2, b2)


def _sc_segmean_src(md, srcp):
    def body(md_hbm, src_hbm, out_hbm,
             msg_v, src_v, den_v, deg_v, idx_v, den_sh, deg_sh):
        c = lax.axis_index("c")
        s = lax.axis_index("s")

        @pl.when(c == 0)
        def _():
            base = s * EPT
            zeros16 = jnp.zeros((16,), jnp.float32)
            iota16 = lax.iota(jnp.int32, 16)
            pltpu.sync_copy(md_hbm.at[pl.ds(base, EPT)], msg_v)
            pltpu.sync_copy(src_hbm.at[pl.ds(base, EPT)], src_v)

            def zero(i, _):
                den_v[pl.ds(i * 16, 16)] = zeros16
                deg_v[pl.ds(i * 16, 16)] = zeros16
                return 0
            lax.fori_loop(0, NP // 16, zero, 0)
            pltpu.sync_copy(den_v.at[pl.ds(0, NROWS_T)],
                            den_sh.at[pl.ds(s * NROWS_T, NROWS_T)])
            pltpu.sync_copy(deg_v.at[pl.ds(0, NROWS_T)],
                            deg_sh.at[pl.ds(s * NROWS_T, NROWS_T)])

            def p(i, _):
                si = src_v[pl.ds(i * 16, 16)]
                mdv = msg_v[pl.ds(i * 16, 16)]
                gid = base + i * 16 + iota16
                w = jnp.where(gid < E, 1.0, 0.0)
                plsc.addupdate_scatter(den_v, [si], mdv)
                plsc.addupdate_scatter(deg_v, [si], w)
                return 0
            lax.fori_loop(0, EPT // 16, p, 0)
            plsc.subcore_barrier()

            def dred(j, _):
                for k in range(8):
                    idx_v[0, pl.ds(k * 16, 16)] = j * 128 + k * 16 + iota16
                pltpu.sync_copy(den_v.at[pl.ds(j * 128, 128)],
                                den_sh.at[idx_v.at[0]], add=True)
                pltpu.sync_copy(deg_v.at[pl.ds(j * 128, 128)],
                                deg_sh.at[idx_v.at[0]], add=True)
                return 0
            lax.fori_loop(0, NP // 128, dred, 0)
            plsc.subcore_barrier()

            pltpu.sync_copy(den_sh.at[pl.ds(s * NROWS_T, NROWS_T)],
                            den_v.at[pl.ds(0, NROWS_T)])
            pltpu.sync_copy(deg_sh.at[pl.ds(s * NROWS_T, NROWS_T)],
                            deg_v.at[pl.ds(0, NROWS_T)])

            def fin(i, _):
                d = den_v[pl.ds(i * 16, 16)]
                g = deg_v[pl.ds(i * 16, 16)]
                den_v[pl.ds(i * 16, 16)] = d / jnp.maximum(g, 1.0)
                return 0
            lax.fori_loop(0, NROWS_T // 16, fin, 0)
            pltpu.sync_copy(den_v.at[pl.ds(0, NROWS_T)],
                            out_hbm.at[pl.ds(s * NROWS_T, NROWS_T)])

    run = pl.kernel(
        body,
        compiler_params=_SC_PARAMS,
        out_type=jax.ShapeDtypeStruct((NP,), jnp.float32),
        mesh=_MESH,
        scratch_types=[
            pltpu.VMEM((EPT,), jnp.float32),
            pltpu.VMEM((EPT,), jnp.int32),
            pltpu.VMEM((NP,), jnp.float32),
            pltpu.VMEM((NP,), jnp.float32),
            pltpu.VMEM((1, 128), jnp.int32),
            pltpu.VMEM_SHARED((NP,), jnp.float32),
            pltpu.VMEM_SHARED((NP,), jnp.float32),
        ],
    )
    return run(md, srcp)


def kernel(features, cluster_features, xws, yws, raw_affine, edge_index, params):
    src = edge_index[0]
    dst = edge_index[1]
    srcp = jnp.pad(src, (0, EP - E))
    dstp = jnp.pad(dst, (0, EP - E))

    parts = [jnp.pad(features, ((0, NP - N), (0, 0))),
             jnp.pad(cluster_features, ((0, NP - N), (0, 0)))]
    bparts = [jnp.zeros((1, 128), jnp.float32)] * 2
    relu_in = False

    for i in range(4):
        w = params['conv%d_W' % i]
        b = params['conv%d_b' % i]
        a_s = params['conv%d_asrc' % i][:, None]
        a_d = params['conv%d_adst' % i][:, None]
        npart = len(parts)
        dp = w.shape[0] // npart
        h4 = w.shape[1] // 8
        wparts = [w[k * dp:(k + 1) * dp] for k in range(npart)]
        *zq, el, er = _gat_linear(parts, wparts, bparts, a_s, a_d, relu_in)
        aggq = _sc_gat_edge(zq, el.reshape(NP), er.reshape(NP), srcp, dstp)
        parts = list(aggq)
        bparts = [b[k * h4:(k + 1) * h4][None, :] for k in range(8)]
        relu_in = True

    xy = jnp.zeros((NP, 16), jnp.float32)
    xy = xy.at[:N, 6].set(xws[:, 0]).at[:N, 7].set(yws[:, 0])
    sxyb = xy.at[:, 0:6].add(params['src_b'][None, :])
    dxyb = xy.at[:, 0:6].add(params['dst_b'][None, :])
    swp = jnp.zeros((128, 16), jnp.float32).at[:, 0:6].set(params['src_W'])
    dwp = jnp.zeros((128, 16), jnp.float32).at[:, 0:6].set(params['dst_W'])
    swparts = [swp[k * 16:(k + 1) * 16] for k in range(8)]
    dwparts = [dwp[k * 16:(k + 1) * 16] for k in range(8)]
    st, dt = _tables(parts, bparts, swparts, dwparts, sxyb, dxyb)

    fa, fb = _sc_gather_tables(st, dt, srcp, dstp)

    rap = jnp.pad(raw_affine, (0, EP - E))[:, None]
    w1 = params['cls1_W']
    w1a = jnp.zeros((16, 8), jnp.float32).at[0:8].set(w1[0:8])
    w1b = jnp.zeros((16, 8), jnp.float32).at[0:8].set(w1[8:16])
    p1 = params['prelu1']
    p1a = jnp.zeros((1, 16), jnp.float32).at[0, 0:8].set(p1[0:8])
    p1b = jnp.zeros((1, 16), jnp.float32).at[0, 0:8].set(p1[8:16])
    pcp, mdp = _mlp(fa, fb, rap, w1a, w1b, params['cls1_b'][None, :],
                    p1a, p1b, params['prelu2'][None, :],
                    params['cls2_W'], params['cls2_b'][None, :])

    pred_den = _sc_segmean_src(mdp.reshape(EP), srcp)
    return pcp[:E], pred_den[:N]
